# Initial kernel scaffold; baseline (speedup 1.0000x reference)
import jax
import jax.numpy as jnp
from jax.experimental import pallas as pl


def kernel(h, p, e, snorm_n, edge_index, atom_emb_0, atom_emb_1, atom_emb_2, atom_emb_3, atom_emb_4, atom_emb_5, atom_emb_6, atom_emb_7, atom_emb_8, bond_emb_0, bond_emb_1, bond_emb_2, Pw, Pb, Aw, Ab, Bw, Bb, Cw, Cb, Dw, Db, Ew, Eb, bnh_g, bnh_b, bne_g, bne_b, W0, b0, W1, b1, W2, b2):
    raise NotImplementedError("write your pallas kernel here")


# speedup vs baseline: 1.5584x; 1.5584x over previous
import functools
import jax
import jax.numpy as jnp
import numpy as np
from jax import lax
from jax.experimental import pallas as pl
from jax.experimental.pallas import tpu as pltpu
from jax.experimental.pallas import tpu_sc as plsc

NN = 10000
NEDGE = 320000
DIM = 128
NLAYER = 4
NC = 2
NS = 16
EC = 80
EPT = NEDGE // NS
NCHUNK = EPT // EC
ROWS_PT = NN // NS
BN_EPS = 1e-5
DEN_EPS = 1e-6

BN_NODE = 1000
BN_GRID = NN // BN_NODE
BE_EDGE = 4000
BE_GRID = NEDGE // BE_EDGE


def _sc_edge_body(last, src2_h, dsts_h, bdtab_h, ehtab_h, ce_h,
                  *refs):
    if last:
        (nd_h,
         srcv, dstsv, bdv, ehv, cev, enewv, statv, zbuf, acc,
         sem1, sem2) = refs
        e2_h = None
        stats_h = None
    else:
        (e2_h, nd_h, stats_h,
         srcv, dstsv, bdv, ehv, cev, enewv, statv, zbuf, acc,
         sem1, sem2) = refs
    c = lax.axis_index("c")
    s = lax.axis_index("s")

    zero = jnp.zeros((16,), jnp.float32)

    def _zrow(i, carry):
        for q in range(8):
            zbuf[i, pl.ds(q * 16, 16)] = zero
        return carry
    lax.fori_loop(0, 40, _zrow, 0)

    @pl.when(s < 10)
    def _zero_acc():
        for kk in range(25):
            pltpu.sync_copy(zbuf, acc.at[pl.ds(s * 1000 + kk * 40, 40)])
    plsc.subcore_barrier()

    def _chunk(k, carry):
        base = s * EPT + k * EC
        pltpu.sync_copy(src2_h.at[pl.ds(c * NEDGE + base, EC)], srcv)
        pltpu.sync_copy(dsts_h.at[pl.ds(base, EC)], dstsv)
        g1 = pltpu.async_copy(bdtab_h.at[srcv], bdv, sem1)
        g2 = pltpu.async_copy(ehtab_h.at[dstsv], ehv, sem2)
        pltpu.sync_copy(ce_h.at[c, pl.ds(base, EC)], cev)
        g1.wait()
        g2.wait()

        def _erow(j, cr):
            cr = list(cr)
            for q in range(4):
                dq = bdv[j, pl.ds(64 + q * 16, 16)]
                ehq = ehv[j, pl.ds(c * 64 + q * 16, 16)]
                ceq = cev[j, pl.ds(q * 16, 16)]
                ev = ceq + dq + ehq
                if not last:
                    enewv[j, pl.ds(q * 16, 16)] = ev
                    cr[q] = cr[q] + ev
                    cr[4 + q] = cr[4 + q] + ev * ev
                sig = 1.0 / (1.0 + jnp.exp(-ev))
                bq = bdv[j, pl.ds(q * 16, 16)]
                bdv[j, pl.ds(q * 16, 16)] = sig * bq
                bdv[j, pl.ds(64 + q * 16, 16)] = sig
            return tuple(cr)

        cr = lax.fori_loop(0, EC, _erow, carry)
        pltpu.sync_copy(bdv, acc.at[dstsv], add=True)
        if not last:
            pltpu.sync_copy(enewv, e2_h.at[c, pl.ds(base, EC)])
        return cr

    init = tuple(zero for _ in range(8))
    st = lax.fori_loop(0, NCHUNK, _chunk, init)

    if not last:
        for q in range(4):
            statv[pl.ds(q * 16, 16)] = st[q]
            statv[pl.ds(64 + q * 16, 16)] = st[4 + q]
        pltpu.sync_copy(statv, stats_h.at[pl.ds((c * NS + s) * 128, 128)])

    plsc.subcore_barrier()

    @pl.when(s < 10)
    def _dump_acc():
        pltpu.sync_copy(acc.at[pl.ds(s * 1000, 1000)],
                        nd_h.at[c, pl.ds(s * 1000, 1000)])


def _sc_edge_pass(bdtab, ehtab, ce2, src2, dst, last):
    mesh = plsc.VectorSubcoreMesh(core_axis_name="c", subcore_axis_name="s",
                                  num_cores=NC, num_subcores=NS)
    if last:
        out_type = (jax.ShapeDtypeStruct((NC, NN, 128), jnp.float32),)
    else:
        out_type = (
            jax.ShapeDtypeStruct((NC, NEDGE, 64), jnp.float32),
            jax.ShapeDtypeStruct((NC, NN, 128), jnp.float32),
            jax.ShapeDtypeStruct((NC * NS * 128,), jnp.float32),
        )
    scratch = [
        pltpu.VMEM((EC,), jnp.int32),
        pltpu.VMEM((EC,), jnp.int32),
        pltpu.VMEM((EC, 128), jnp.float32),
        pltpu.VMEM((EC, 128), jnp.float32),
        pltpu.VMEM((EC, 64), jnp.float32),
        pltpu.VMEM((EC, 64), jnp.float32),
        pltpu.VMEM((128,), jnp.float32),
        pltpu.VMEM((40, 128), jnp.float32),
        pltpu.VMEM_SHARED((NN, 128), jnp.float32),
        pltpu.SemaphoreType.DMA,
        pltpu.SemaphoreType.DMA,
    ]
    fn = pl.kernel(functools.partial(_sc_edge_body, last),
                   out_type=out_type, mesh=mesh, scratch_types=scratch)
    return fn(src2, dst, bdtab, ehtab, ce2)


def _node_mats(hf, wa, ab, wb, bb, wd, db, we, eb,
               hf_out, ah_out, bd_out, eh_out):
    if hf_out is not None:
        hf_out[...] = hf
    f32 = jnp.float32
    Ah = jnp.dot(hf, wa[...], preferred_element_type=f32) + ab[...]
    Bh = jnp.dot(hf, wb[...], preferred_element_type=f32) + bb[...]
    Dh = jnp.dot(hf, wd[...], preferred_element_type=f32) + db[...]
    Eh = jnp.dot(hf, we[...], preferred_element_type=f32) + eb[...]
    ah_out[...] = Ah
    bd_out[0] = jnp.concatenate([Bh[:, :64], Dh[:, :64]], axis=1)
    bd_out[1] = jnp.concatenate([Bh[:, 64:], Dh[:, 64:]], axis=1)
    eh_out[...] = Eh


def _enc_node_body(h_ref, p_ref, acat, pw, pb2,
                   wa, ab, wb, bb, wd, db, we, eb,
                   hf_out, ah_out, bd_out, eh_out):
    f32 = jnp.float32
    hf = jnp.dot(p_ref[...], pw[...], preferred_element_type=f32) + pb2[...]
    iot = lax.broadcasted_iota(jnp.int32, (1, 128), 1)
    for i in range(9):
        oh = (h_ref[:, i:i + 1] == iot).astype(f32)
        hf = hf + jnp.dot(oh, acat[i], preferred_element_type=f32)
    _node_mats(hf, wa, ab, wb, bb, wd, db, we, eb,
               hf_out, ah_out, bd_out, eh_out)


def _enc_node_call(h, p, acat, pw, pb2, wa, ab, wb, bb, wd, db, we, eb):
    B = BN_NODE
    full = lambda *shape: pl.BlockSpec(shape, lambda i: (0,) * len(shape))
    blk = lambda *shape: pl.BlockSpec(shape, lambda i: (i,) + (0,) * (len(shape) - 1))
    blk2 = lambda *shape: pl.BlockSpec(shape, lambda i: (0, i) + (0,) * (len(shape) - 2))
    return pl.pallas_call(
        _enc_node_body,
        grid=(BN_GRID,),
        in_specs=[blk(B, 9), blk(B, 8), full(9, 128, 128), full(8, 128),
                  full(1, 128)] + [full(128, 128), full(1, 128)] * 4,
        out_specs=[blk(B, 128), blk(B, 128), blk2(2, B, 128), blk(B, 128)],
        out_shape=[
            jax.ShapeDtypeStruct((NN, 128), jnp.float32),
            jax.ShapeDtypeStruct((NN, 128), jnp.float32),
            jax.ShapeDtypeStruct((2, NN, 128), jnp.float32),
            jax.ShapeDtypeStruct((NN, 128), jnp.float32),
        ],
    )(h, p, acat, pw, pb2, wa, ab, wb, bb, wd, db, we, eb)


def _bond_edge_body(e_ref, bcat, cw, cb2, ef_out, ce_out):
    f32 = jnp.float32
    iot = lax.broadcasted_iota(jnp.int32, (1, 8), 1)
    ef = jnp.zeros((e_ref.shape[0], 128), f32)
    for i in range(3):
        oh = (e_ref[:, i:i + 1] == iot).astype(f32)
        ef = ef + jnp.dot(oh, bcat[i], preferred_element_type=f32)
    ef_out[...] = ef
    ce = jnp.dot(ef, cw[...], preferred_element_type=f32) + cb2[...]
    ce_out[0] = ce[:, :64]
    ce_out[1] = ce[:, 64:]


def _bond_edge_call(e, bcat, cw, cb2):
    B = BE_EDGE
    full = lambda *shape: pl.BlockSpec(shape, lambda i: (0,) * len(shape))
    blk = lambda *shape: pl.BlockSpec(shape, lambda i: (i,) + (0,) * (len(shape) - 1))
    blk2 = lambda *shape: pl.BlockSpec(shape, lambda i: (0, i) + (0,) * (len(shape) - 2))
    return pl.pallas_call(
        _bond_edge_body,
        grid=(BE_GRID,),
        in_specs=[blk(B, 3), full(3, 8, 128), full(128, 128), full(1, 128)],
        out_specs=[blk(B, 128), blk2(2, B, 64)],
        out_shape=[
            jax.ShapeDtypeStruct((NEDGE, 128), jnp.float32),
            jax.ShapeDtypeStruct((2, NEDGE, 64), jnp.float32),
        ],
    )(e, bcat, cw, cb2)


def _hnew_body(nd_ref, ah_ref, hn_out, ps_out):
    num = jnp.concatenate([nd_ref[0][:, :64], nd_ref[1][:, :64]], axis=1)
    den = jnp.concatenate([nd_ref[0][:, 64:], nd_ref[1][:, 64:]], axis=1)
    hn = ah_ref[...] + num / (den + DEN_EPS)
    hn_out[...] = hn
    s1 = jnp.sum(hn, axis=0, keepdims=True)
    s2 = jnp.sum(hn * hn, axis=0, keepdims=True)
    ps_out[...] = jnp.concatenate(
        [s1, s2, jnp.zeros((6, 128), jnp.float32)], axis=0).reshape(1, 8, 128)


def _hnew_call(nd, ah):
    B = BN_NODE
    blk = lambda *shape: pl.BlockSpec(shape, lambda i: (i,) + (0,) * (len(shape) - 1))
    blk2 = lambda *shape: pl.BlockSpec(shape, lambda i: (0, i) + (0,) * (len(shape) - 2))
    return pl.pallas_call(
        _hnew_body,
        grid=(BN_GRID,),
        in_specs=[blk2(2, B, 128), blk(B, 128)],
        out_specs=[blk(B, 128), blk(1, 8, 128)],
        out_shape=[
            jax.ShapeDtypeStruct((NN, 128), jnp.float32),
            jax.ShapeDtypeStruct((BN_GRID, 8, 128), jnp.float32),
        ],
    )(nd, ah)


def _node_stats(ps_ref):
    m = jnp.sum(ps_ref[:, 0, :], axis=0, keepdims=True) / NN
    v = jnp.sum(ps_ref[:, 1, :], axis=0, keepdims=True) / NN - m * m
    return m, v


def _nodeup_body(hn_ref, hfp_ref, ps_ref, g2, b2,
                 wa, ab, wb, bb, wd, db, we, eb,
                 hf_out, ah_out, bd_out, eh_out):
    m, v = _node_stats(ps_ref)
    bn = g2[...] * (hn_ref[...] - m) * lax.rsqrt(v + BN_EPS) + b2[...]
    hf = hfp_ref[...] + jnp.maximum(bn, 0.0)
    _node_mats(hf, wa, ab, wb, bb, wd, db, we, eb,
               hf_out, ah_out, bd_out, eh_out)


def _nodeup_call(hn, hfp, ps, g2, b2, wa, ab, wb, bb, wd, db, we, eb):
    B = BN_NODE
    full = lambda *shape: pl.BlockSpec(shape, lambda i: (0,) * len(shape))
    blk = lambda *shape: pl.BlockSpec(shape, lambda i: (i,) + (0,) * (len(shape) - 1))
    blk2 = lambda *shape: pl.BlockSpec(shape, lambda i: (0, i) + (0,) * (len(shape) - 2))
    return pl.pallas_call(
        _nodeup_body,
        grid=(BN_GRID,),
        in_specs=[blk(B, 128), blk(B, 128), full(BN_GRID, 8, 128),
                  full(1, 128), full(1, 128)]
                 + [full(128, 128), full(1, 128)] * 4,
        out_specs=[blk(B, 128), blk(B, 128), blk2(2, B, 128), blk(B, 128)],
        out_shape=[
            jax.ShapeDtypeStruct((NN, 128), jnp.float32),
            jax.ShapeDtypeStruct((NN, 128), jnp.float32),
            jax.ShapeDtypeStruct((2, NN, 128), jnp.float32),
            jax.ShapeDtypeStruct((NN, 128), jnp.float32),
        ],
    )(hn, hfp, ps, g2, b2, wa, ab, wb, bb, wd, db, we, eb)


def _edge_stats(est_ref):
    e0 = jnp.sum(est_ref[0], axis=0)
    e1 = jnp.sum(est_ref[1], axis=0)
    m = jnp.concatenate([e0[:64], e1[:64]]).reshape(1, 128) / NEDGE
    q = jnp.concatenate([e0[64:], e1[64:]]).reshape(1, 128) / NEDGE
    return m, q - m * m


def _edgeup_body(write_ef, e2_ref, ef_ref, est_ref, g2, b2, cw, cb2, *outs):
    if write_ef:
        ef_out, ce_out = outs
    else:
        (ce_out,) = outs
    m, v = _edge_stats(est_ref)
    en = jnp.concatenate([e2_ref[0], e2_ref[1]], axis=1)
    bn = g2[...] * (en - m) * lax.rsqrt(v + BN_EPS) + b2[...]
    ef = ef_ref[...] + jnp.maximum(bn, 0.0)
    if write_ef:
        ef_out[...] = ef
    ce = jnp.dot(ef, cw[...], preferred_element_type=jnp.float32) + cb2[...]
    ce_out[0] = ce[:, :64]
    ce_out[1] = ce[:, 64:]


def _edgeup_call(e2, ef, est, g2, b2, cw, cb2, write_ef):
    B = BE_EDGE
    full = lambda *shape: pl.BlockSpec(shape, lambda i: (0,) * len(shape))
    blk = lambda *shape: pl.BlockSpec(shape, lambda i: (i,) + (0,) * (len(shape) - 1))
    blk2 = lambda *shape: pl.BlockSpec(shape, lambda i: (0, i) + (0,) * (len(shape) - 2))
    out_specs = [blk2(2, B, 64)]
    out_shape = [jax.ShapeDtypeStruct((2, NEDGE, 64), jnp.float32)]
    if write_ef:
        out_specs = [blk(B, 128)] + out_specs
        out_shape = [jax.ShapeDtypeStruct((NEDGE, 128), jnp.float32)] + out_shape
    return pl.pallas_call(
        functools.partial(_edgeup_body, write_ef),
        grid=(BE_GRID,),
        in_specs=[blk2(2, B, 64), blk(B, 128), full(2, NS, 128),
                  full(1, 128), full(1, 128), full(128, 128), full(1, 128)],
        out_specs=out_specs,
        out_shape=out_shape,
    )(e2, ef, est, g2, b2, cw, cb2)


def _final_body(hn_ref, hfp_ref, ps_ref, g2, b2,
                w0, b0r, w1, b1r, w2, b2r, y_out):
    m, v = _node_stats(ps_ref)
    bn = g2[...] * (hn_ref[...] - m) * lax.rsqrt(v + BN_EPS) + b2[...]
    hf = hfp_ref[...] + jnp.maximum(bn, 0.0)
    hg = jnp.sum(hf, axis=0, keepdims=True) / NN
    f32 = jnp.float32
    y = jnp.maximum(jnp.dot(hg, w0[...], preferred_element_type=f32) + b0r[...], 0.0)
    y = jnp.maximum(jnp.dot(y, w1[...], preferred_element_type=f32) + b1r[...], 0.0)
    y = jnp.dot(y, w2[...], preferred_element_type=f32) + b2r[...]
    y_out[...] = y


def _final_call(hn, hfp, ps, g2, b2, w0, b0r, w1, b1r, w2, b2r):
    full = lambda *shape: pl.BlockSpec(shape, lambda i: (0,) * len(shape))
    return pl.pallas_call(
        _final_body,
        grid=(1,),
        in_specs=[full(NN, 128), full(NN, 128), full(BN_GRID, 8, 128),
                  full(1, 128), full(1, 128), full(128, 64), full(1, 64),
                  full(64, 32), full(1, 32), full(32, 128), full(1, 128)],
        out_specs=[full(1, 128)],
        out_shape=[jax.ShapeDtypeStruct((1, 128), jnp.float32)],
    )(hn, hfp, ps, g2, b2, w0, b0r, w1, b1r, w2, b2r)


def kernel(h, p, e, snorm_n, edge_index,
           atom_emb_0, atom_emb_1, atom_emb_2, atom_emb_3, atom_emb_4,
           atom_emb_5, atom_emb_6, atom_emb_7, atom_emb_8,
           bond_emb_0, bond_emb_1, bond_emb_2,
           Pw, Pb, Aw, Ab, Bw, Bb, Cw, Cb, Dw, Db, Ew, Eb,
           bnh_g, bnh_b, bne_g, bne_b, W0, b0, W1, b1, W2, b2):
    f32 = jnp.float32
    atom_tabs = [atom_emb_0, atom_emb_1, atom_emb_2, atom_emb_3, atom_emb_4,
                 atom_emb_5, atom_emb_6, atom_emb_7, atom_emb_8]
    acat = jnp.stack([jnp.pad(t, ((0, 128 - t.shape[0]), (0, 0)))
                      for t in atom_tabs])
    bond_tabs = [bond_emb_0, bond_emb_1, bond_emb_2]
    bcat = jnp.stack([jnp.pad(t, ((0, 8 - t.shape[0]), (0, 0)))
                      for t in bond_tabs])
    r1 = lambda x: x.reshape(1, -1).astype(f32)

    src = edge_index[0].astype(jnp.int32)
    dst = edge_index[1].astype(jnp.int32)
    src2 = jnp.concatenate([src, src + NN])

    h = h.astype(jnp.int32)
    e = e.astype(jnp.int32)

    hf, ah, bd, eh = _enc_node_call(
        h, p.astype(f32), acat, Pw, r1(Pb),
        Aw[0], r1(Ab[0]), Bw[0], r1(Bb[0]), Dw[0], r1(Db[0]), Ew[0], r1(Eb[0]))
    ef, ce2 = _bond_edge_call(e, bcat, Cw[0], r1(Cb[0]))

    for l in range(NLAYER):
        last = (l == NLAYER - 1)
        bdtab = bd.reshape(2 * NN, 128)
        if last:
            (nd,) = _sc_edge_pass(bdtab, eh, ce2, src2, dst, True)
        else:
            e2, nd, est = _sc_edge_pass(bdtab, eh, ce2, src2, dst, False)
        hn, ps = _hnew_call(nd, ah)
        if last:
            (y,) = _final_call(hn, hf, ps, r1(bnh_g[l]), r1(bnh_b[l]),
                               W0, r1(b0), W1, r1(b1), W2, r1(b2))
        else:
            hf, ah, bd, eh = _nodeup_call(
                hn, hf, ps, r1(bnh_g[l]), r1(bnh_b[l]),
                Aw[l + 1], r1(Ab[l + 1]), Bw[l + 1], r1(Bb[l + 1]),
                Dw[l + 1], r1(Db[l + 1]), Ew[l + 1], r1(Eb[l + 1]))
            est3 = est.reshape(2, NS, 128)
            if l < NLAYER - 2:
                ef, ce2 = _edgeup_call(e2, ef, est3, r1(bne_g[l]),
                                       r1(bne_b[l]), Cw[l + 1], r1(Cb[l + 1]),
                                       True)
            else:
                (ce2,) = _edgeup_call(e2, ef, est3, r1(bne_g[l]),
                                      r1(bne_b[l]), Cw[l + 1], r1(Cb[l + 1]),
                                      False)
    return y


# speedup vs baseline: 1.7830x; 1.1442x over previous
import functools
import jax
import jax.numpy as jnp
import numpy as np
from jax import lax
from jax.experimental import pallas as pl
from jax.experimental.pallas import tpu as pltpu
from jax.experimental.pallas import tpu_sc as plsc

NN = 10000
NEDGE = 320000
DIM = 128
NLAYER = 4
NC = 2
NS = 16
EC = 40
EPT = NEDGE // NS
NCHUNK = EPT // EC
NHALF = NCHUNK // 2
ROWS_PT = NN // NS
BN_EPS = 1e-5
DEN_EPS = 1e-6

BN_NODE = 1000
BN_GRID = NN // BN_NODE
BE_EDGE = 4000
BE_GRID = NEDGE // BE_EDGE


def _sc_edge_body(last, icat_h, bdtab_h, ehtab_h, ce_h,
                  *refs):
    if last:
        (nd_h,
         idxv0, idxv1, bdv0, bdv1, ehv0, ehv1, cev0, cev1,
         env0, env1, statv, zbuf, acc,
         sbd0, sbd1, seh0, seh1, sce0, sce1) = refs
        e2_h = None
        stats_h = None
    else:
        (e2_h, nd_h, stats_h,
         idxv0, idxv1, bdv0, bdv1, ehv0, ehv1, cev0, cev1,
         env0, env1, statv, zbuf, acc,
         sbd0, sbd1, seh0, seh1, sce0, sce1) = refs
    c = lax.axis_index("c")
    s = lax.axis_index("s")

    zero = jnp.zeros((16,), jnp.float32)

    def _zrow(i, carry):
        for q in range(8):
            zbuf[i, pl.ds(q * 16, 16)] = zero
        return carry
    lax.fori_loop(0, 40, _zrow, 0)

    @pl.when(s < 10)
    def _zero_acc():
        for kk in range(25):
            pltpu.sync_copy(zbuf, acc.at[pl.ds(s * 1000 + kk * 40, 40)])
    plsc.subcore_barrier()

    def _load_idx(k, idxv):
        pltpu.sync_copy(icat_h.at[c * (NS * NCHUNK) + s * NCHUNK + k], idxv)

    def _issue(k, idxv, bdv, ehv, cev, sbd, seh, sce):
        base = s * EPT + k * EC
        pltpu.async_copy(bdtab_h.at[idxv.at[0]], bdv, sbd)
        pltpu.async_copy(ehtab_h.at[idxv.at[1]], ehv, seh)
        pltpu.async_copy(ce_h.at[c, pl.ds(base, EC)], cev, sce)

    def _drain(bdv, ehv, cev, sbd, seh, sce):
        pltpu.make_async_copy(bdtab_h.at[idxv0.at[0]], bdv, sbd).wait()
        pltpu.make_async_copy(ehtab_h.at[idxv0.at[1]], ehv, seh).wait()
        pltpu.make_async_copy(ce_h.at[c, pl.ds(0, EC)], cev, sce).wait()

    def _compute(k, idxv, bdv, ehv, cev, env, cr):
        def _erow(j, cr2):
            cr2 = list(cr2)
            for q in range(4):
                dq = bdv[j, pl.ds(64 + q * 16, 16)]
                ehq = ehv[j, pl.ds(c * 64 + q * 16, 16)]
                ceq = cev[j, pl.ds(q * 16, 16)]
                ev = ceq + dq + ehq
                if not last:
                    env[j, pl.ds(q * 16, 16)] = ev
                    cr2[q] = cr2[q] + ev
                    cr2[4 + q] = cr2[4 + q] + ev * ev
                sig = 1.0 / (1.0 + jnp.exp(-ev))
                bq = bdv[j, pl.ds(q * 16, 16)]
                bdv[j, pl.ds(q * 16, 16)] = sig * bq
                bdv[j, pl.ds(64 + q * 16, 16)] = sig
            return tuple(cr2)

        cr = lax.fori_loop(0, EC, _erow, cr)
        pltpu.sync_copy(bdv, acc.at[idxv.at[1]], add=True)
        if not last:
            base = s * EPT + k * EC
            pltpu.sync_copy(env, e2_h.at[c, pl.ds(base, EC)])
        return cr

    _load_idx(0, idxv0)
    _issue(0, idxv0, bdv0, ehv0, cev0, sbd0, seh0, sce0)

    def _pair(kp, cr):
        ka = 2 * kp
        _load_idx(ka + 1, idxv1)
        _issue(ka + 1, idxv1, bdv1, ehv1, cev1, sbd1, seh1, sce1)
        _drain(bdv0, ehv0, cev0, sbd0, seh0, sce0)
        cr = _compute(ka, idxv0, bdv0, ehv0, cev0, env0, cr)

        @pl.when(kp + 1 < NHALF)
        def _prefetch_next():
            _load_idx(ka + 2, idxv0)
            _issue(ka + 2, idxv0, bdv0, ehv0, cev0, sbd0, seh0, sce0)
        _drain(bdv1, ehv1, cev1, sbd1, seh1, sce1)
        cr = _compute(ka + 1, idxv1, bdv1, ehv1, cev1, env1, cr)
        return cr

    init = tuple(zero for _ in range(8))
    st = lax.fori_loop(0, NHALF, _pair, init)

    if not last:
        for q in range(4):
            statv[pl.ds(q * 16, 16)] = st[q]
            statv[pl.ds(64 + q * 16, 16)] = st[4 + q]
        pltpu.sync_copy(statv, stats_h.at[pl.ds((c * NS + s) * 128, 128)])

    plsc.subcore_barrier()

    @pl.when(s < 10)
    def _dump_acc():
        pltpu.sync_copy(acc.at[pl.ds(s * 1000, 1000)],
                        nd_h.at[c, pl.ds(s * 1000, 1000)])


def _sc_edge_pass(bdtab, ehtab, ce2, icat, last):
    mesh = plsc.VectorSubcoreMesh(core_axis_name="c", subcore_axis_name="s",
                                  num_cores=NC, num_subcores=NS)
    if last:
        out_type = (jax.ShapeDtypeStruct((NC, NN, 128), jnp.float32),)
    else:
        out_type = (
            jax.ShapeDtypeStruct((NC, NEDGE, 64), jnp.float32),
            jax.ShapeDtypeStruct((NC, NN, 128), jnp.float32),
            jax.ShapeDtypeStruct((NC * NS * 128,), jnp.float32),
        )
    scratch = [
        pltpu.VMEM((2, EC), jnp.int32),
        pltpu.VMEM((2, EC), jnp.int32),
        pltpu.VMEM((EC, 128), jnp.float32),
        pltpu.VMEM((EC, 128), jnp.float32),
        pltpu.VMEM((EC, 128), jnp.float32),
        pltpu.VMEM((EC, 128), jnp.float32),
        pltpu.VMEM((EC, 64), jnp.float32),
        pltpu.VMEM((EC, 64), jnp.float32),
        pltpu.VMEM((EC, 64), jnp.float32),
        pltpu.VMEM((EC, 64), jnp.float32),
        pltpu.VMEM((128,), jnp.float32),
        pltpu.VMEM((40, 128), jnp.float32),
        pltpu.VMEM_SHARED((NN, 128), jnp.float32),
    ] + [pltpu.SemaphoreType.DMA] * 6
    fn = pl.kernel(functools.partial(_sc_edge_body, last),
                   out_type=out_type, mesh=mesh, scratch_types=scratch)
    return fn(icat, bdtab, ehtab, ce2)


def _node_mats(hf, wa, ab, wb, bb, wd, db, we, eb,
               hf_out, ah_out, bd_out, eh_out):
    if hf_out is not None:
        hf_out[...] = hf
    f32 = jnp.float32
    Ah = jnp.dot(hf, wa[...], preferred_element_type=f32) + ab[...]
    Bh = jnp.dot(hf, wb[...], preferred_element_type=f32) + bb[...]
    Dh = jnp.dot(hf, wd[...], preferred_element_type=f32) + db[...]
    Eh = jnp.dot(hf, we[...], preferred_element_type=f32) + eb[...]
    ah_out[...] = Ah
    bd_out[0] = jnp.concatenate([Bh[:, :64], Dh[:, :64]], axis=1)
    bd_out[1] = jnp.concatenate([Bh[:, 64:], Dh[:, 64:]], axis=1)
    eh_out[...] = Eh


def _enc_node_body(h_ref, p_ref, acat, pw, pb2,
                   wa, ab, wb, bb, wd, db, we, eb,
                   hf_out, ah_out, bd_out, eh_out):
    f32 = jnp.float32
    hf = jnp.dot(p_ref[...], pw[...], preferred_element_type=f32) + pb2[...]
    iot = lax.broadcasted_iota(jnp.int32, (1, 128), 1)
    for i in range(9):
        oh = (h_ref[:, i:i + 1] == iot).astype(f32)
        hf = hf + jnp.dot(oh, acat[i], preferred_element_type=f32)
    _node_mats(hf, wa, ab, wb, bb, wd, db, we, eb,
               hf_out, ah_out, bd_out, eh_out)


def _enc_node_call(h, p, acat, pw, pb2, wa, ab, wb, bb, wd, db, we, eb):
    B = BN_NODE
    full = lambda *shape: pl.BlockSpec(shape, lambda i: (0,) * len(shape))
    blk = lambda *shape: pl.BlockSpec(shape, lambda i: (i,) + (0,) * (len(shape) - 1))
    blk2 = lambda *shape: pl.BlockSpec(shape, lambda i: (0, i) + (0,) * (len(shape) - 2))
    return pl.pallas_call(
        _enc_node_body,
        grid=(BN_GRID,),
        in_specs=[blk(B, 9), blk(B, 8), full(9, 128, 128), full(8, 128),
                  full(1, 128)] + [full(128, 128), full(1, 128)] * 4,
        out_specs=[blk(B, 128), blk(B, 128), blk2(2, B, 128), blk(B, 128)],
        out_shape=[
            jax.ShapeDtypeStruct((NN, 128), jnp.float32),
            jax.ShapeDtypeStruct((NN, 128), jnp.float32),
            jax.ShapeDtypeStruct((2, NN, 128), jnp.float32),
            jax.ShapeDtypeStruct((NN, 128), jnp.float32),
        ],
    )(h, p, acat, pw, pb2, wa, ab, wb, bb, wd, db, we, eb)


def _bond_edge_body(e_ref, bcat, cw, cb2, ef_out, ce_out):
    f32 = jnp.float32
    iot = lax.broadcasted_iota(jnp.int32, (1, 8), 1)
    ef = jnp.zeros((e_ref.shape[0], 128), f32)
    for i in range(3):
        oh = (e_ref[:, i:i + 1] == iot).astype(f32)
        ef = ef + jnp.dot(oh, bcat[i], preferred_element_type=f32)
    ef_out[...] = ef
    ce = jnp.dot(ef, cw[...], preferred_element_type=f32) + cb2[...]
    ce_out[0] = ce[:, :64]
    ce_out[1] = ce[:, 64:]


def _bond_edge_call(e, bcat, cw, cb2):
    B = BE_EDGE
    full = lambda *shape: pl.BlockSpec(shape, lambda i: (0,) * len(shape))
    blk = lambda *shape: pl.BlockSpec(shape, lambda i: (i,) + (0,) * (len(shape) - 1))
    blk2 = lambda *shape: pl.BlockSpec(shape, lambda i: (0, i) + (0,) * (len(shape) - 2))
    return pl.pallas_call(
        _bond_edge_body,
        grid=(BE_GRID,),
        in_specs=[blk(B, 3), full(3, 8, 128), full(128, 128), full(1, 128)],
        out_specs=[blk(B, 128), blk2(2, B, 64)],
        out_shape=[
            jax.ShapeDtypeStruct((NEDGE, 128), jnp.float32),
            jax.ShapeDtypeStruct((2, NEDGE, 64), jnp.float32),
        ],
    )(e, bcat, cw, cb2)


def _hnew_body(nd_ref, ah_ref, hn_out, ps_out):
    num = jnp.concatenate([nd_ref[0][:, :64], nd_ref[1][:, :64]], axis=1)
    den = jnp.concatenate([nd_ref[0][:, 64:], nd_ref[1][:, 64:]], axis=1)
    hn = ah_ref[...] + num / (den + DEN_EPS)
    hn_out[...] = hn
    s1 = jnp.sum(hn, axis=0, keepdims=True)
    s2 = jnp.sum(hn * hn, axis=0, keepdims=True)
    ps_out[...] = jnp.concatenate(
        [s1, s2, jnp.zeros((6, 128), jnp.float32)], axis=0).reshape(1, 8, 128)


def _hnew_call(nd, ah):
    B = BN_NODE
    blk = lambda *shape: pl.BlockSpec(shape, lambda i: (i,) + (0,) * (len(shape) - 1))
    blk2 = lambda *shape: pl.BlockSpec(shape, lambda i: (0, i) + (0,) * (len(shape) - 2))
    return pl.pallas_call(
        _hnew_body,
        grid=(BN_GRID,),
        in_specs=[blk2(2, B, 128), blk(B, 128)],
        out_specs=[blk(B, 128), blk(1, 8, 128)],
        out_shape=[
            jax.ShapeDtypeStruct((NN, 128), jnp.float32),
            jax.ShapeDtypeStruct((BN_GRID, 8, 128), jnp.float32),
        ],
    )(nd, ah)


def _node_stats(ps_ref):
    m = jnp.sum(ps_ref[:, 0, :], axis=0, keepdims=True) / NN
    v = jnp.sum(ps_ref[:, 1, :], axis=0, keepdims=True) / NN - m * m
    return m, v


def _nodeup_body(hn_ref, hfp_ref, ps_ref, g2, b2,
                 wa, ab, wb, bb, wd, db, we, eb,
                 hf_out, ah_out, bd_out, eh_out):
    m, v = _node_stats(ps_ref)
    bn = g2[...] * (hn_ref[...] - m) * lax.rsqrt(v + BN_EPS) + b2[...]
    hf = hfp_ref[...] + jnp.maximum(bn, 0.0)
    _node_mats(hf, wa, ab, wb, bb, wd, db, we, eb,
               hf_out, ah_out, bd_out, eh_out)


def _nodeup_call(hn, hfp, ps, g2, b2, wa, ab, wb, bb, wd, db, we, eb):
    B = BN_NODE
    full = lambda *shape: pl.BlockSpec(shape, lambda i: (0,) * len(shape))
    blk = lambda *shape: pl.BlockSpec(shape, lambda i: (i,) + (0,) * (len(shape) - 1))
    blk2 = lambda *shape: pl.BlockSpec(shape, lambda i: (0, i) + (0,) * (len(shape) - 2))
    return pl.pallas_call(
        _nodeup_body,
        grid=(BN_GRID,),
        in_specs=[blk(B, 128), blk(B, 128), full(BN_GRID, 8, 128),
                  full(1, 128), full(1, 128)]
                 + [full(128, 128), full(1, 128)] * 4,
        out_specs=[blk(B, 128), blk(B, 128), blk2(2, B, 128), blk(B, 128)],
        out_shape=[
            jax.ShapeDtypeStruct((NN, 128), jnp.float32),
            jax.ShapeDtypeStruct((NN, 128), jnp.float32),
            jax.ShapeDtypeStruct((2, NN, 128), jnp.float32),
            jax.ShapeDtypeStruct((NN, 128), jnp.float32),
        ],
    )(hn, hfp, ps, g2, b2, wa, ab, wb, bb, wd, db, we, eb)


def _edge_stats(est_ref):
    e0 = jnp.sum(est_ref[0], axis=0)
    e1 = jnp.sum(est_ref[1], axis=0)
    m = jnp.concatenate([e0[:64], e1[:64]]).reshape(1, 128) / NEDGE
    q = jnp.concatenate([e0[64:], e1[64:]]).reshape(1, 128) / NEDGE
    return m, q - m * m


def _edgeup_body(write_ef, e2_ref, ef_ref, est_ref, g2, b2, cw, cb2, *outs):
    if write_ef:
        ef_out, ce_out = outs
    else:
        (ce_out,) = outs
    m, v = _edge_stats(est_ref)
    en = jnp.concatenate([e2_ref[0], e2_ref[1]], axis=1)
    bn = g2[...] * (en - m) * lax.rsqrt(v + BN_EPS) + b2[...]
    ef = ef_ref[...] + jnp.maximum(bn, 0.0)
    if write_ef:
        ef_out[...] = ef
    ce = jnp.dot(ef, cw[...], preferred_element_type=jnp.float32) + cb2[...]
    ce_out[0] = ce[:, :64]
    ce_out[1] = ce[:, 64:]


def _edgeup_call(e2, ef, est, g2, b2, cw, cb2, write_ef):
    B = BE_EDGE
    full = lambda *shape: pl.BlockSpec(shape, lambda i: (0,) * len(shape))
    blk = lambda *shape: pl.BlockSpec(shape, lambda i: (i,) + (0,) * (len(shape) - 1))
    blk2 = lambda *shape: pl.BlockSpec(shape, lambda i: (0, i) + (0,) * (len(shape) - 2))
    out_specs = [blk2(2, B, 64)]
    out_shape = [jax.ShapeDtypeStruct((2, NEDGE, 64), jnp.float32)]
    if write_ef:
        out_specs = [blk(B, 128)] + out_specs
        out_shape = [jax.ShapeDtypeStruct((NEDGE, 128), jnp.float32)] + out_shape
    return pl.pallas_call(
        functools.partial(_edgeup_body, write_ef),
        grid=(BE_GRID,),
        in_specs=[blk2(2, B, 64), blk(B, 128), full(2, NS, 128),
                  full(1, 128), full(1, 128), full(128, 128), full(1, 128)],
        out_specs=out_specs,
        out_shape=out_shape,
    )(e2, ef, est, g2, b2, cw, cb2)


def _final_body(hn_ref, hfp_ref, ps_ref, g2, b2,
                w0, b0r, w1, b1r, w2, b2r, y_out):
    m, v = _node_stats(ps_ref)
    bn = g2[...] * (hn_ref[...] - m) * lax.rsqrt(v + BN_EPS) + b2[...]
    hf = hfp_ref[...] + jnp.maximum(bn, 0.0)
    hg = jnp.sum(hf, axis=0, keepdims=True) / NN
    f32 = jnp.float32
    y = jnp.maximum(jnp.dot(hg, w0[...], preferred_element_type=f32) + b0r[...], 0.0)
    y = jnp.maximum(jnp.dot(y, w1[...], preferred_element_type=f32) + b1r[...], 0.0)
    y = jnp.dot(y, w2[...], preferred_element_type=f32) + b2r[...]
    y_out[...] = y


def _final_call(hn, hfp, ps, g2, b2, w0, b0r, w1, b1r, w2, b2r):
    full = lambda *shape: pl.BlockSpec(shape, lambda i: (0,) * len(shape))
    return pl.pallas_call(
        _final_body,
        grid=(1,),
        in_specs=[full(NN, 128), full(NN, 128), full(BN_GRID, 8, 128),
                  full(1, 128), full(1, 128), full(128, 64), full(1, 64),
                  full(64, 32), full(1, 32), full(32, 128), full(1, 128)],
        out_specs=[full(1, 128)],
        out_shape=[jax.ShapeDtypeStruct((1, 128), jnp.float32)],
    )(hn, hfp, ps, g2, b2, w0, b0r, w1, b1r, w2, b2r)


def kernel(h, p, e, snorm_n, edge_index,
           atom_emb_0, atom_emb_1, atom_emb_2, atom_emb_3, atom_emb_4,
           atom_emb_5, atom_emb_6, atom_emb_7, atom_emb_8,
           bond_emb_0, bond_emb_1, bond_emb_2,
           Pw, Pb, Aw, Ab, Bw, Bb, Cw, Cb, Dw, Db, Ew, Eb,
           bnh_g, bnh_b, bne_g, bne_b, W0, b0, W1, b1, W2, b2):
    f32 = jnp.float32
    atom_tabs = [atom_emb_0, atom_emb_1, atom_emb_2, atom_emb_3, atom_emb_4,
                 atom_emb_5, atom_emb_6, atom_emb_7, atom_emb_8]
    acat = jnp.stack([jnp.pad(t, ((0, 128 - t.shape[0]), (0, 0)))
                      for t in atom_tabs])
    bond_tabs = [bond_emb_0, bond_emb_1, bond_emb_2]
    bcat = jnp.stack([jnp.pad(t, ((0, 8 - t.shape[0]), (0, 0)))
                      for t in bond_tabs])
    r1 = lambda x: x.reshape(1, -1).astype(f32)

    src = edge_index[0].astype(jnp.int32)
    dst = edge_index[1].astype(jnp.int32)
    src2 = jnp.concatenate([src, src + NN])
    src2r = src2.reshape(2, NS, NCHUNK, EC)
    dstr = jnp.broadcast_to(dst.reshape(1, NS, NCHUNK, EC),
                            (2, NS, NCHUNK, EC))
    icat = jnp.stack([src2r, dstr], axis=3).reshape(2 * NS * NCHUNK, 2, EC)

    h = h.astype(jnp.int32)
    e = e.astype(jnp.int32)

    hf, ah, bd, eh = _enc_node_call(
        h, p.astype(f32), acat, Pw, r1(Pb),
        Aw[0], r1(Ab[0]), Bw[0], r1(Bb[0]), Dw[0], r1(Db[0]), Ew[0], r1(Eb[0]))
    ef, ce2 = _bond_edge_call(e, bcat, Cw[0], r1(Cb[0]))

    for l in range(NLAYER):
        last = (l == NLAYER - 1)
        bdtab = bd.reshape(2 * NN, 128)
        if last:
            (nd,) = _sc_edge_pass(bdtab, eh, ce2, icat, True)
        else:
            e2, nd, est = _sc_edge_pass(bdtab, eh, ce2, icat, False)
        hn, ps = _hnew_call(nd, ah)
        if last:
            (y,) = _final_call(hn, hf, ps, r1(bnh_g[l]), r1(bnh_b[l]),
                               W0, r1(b0), W1, r1(b1), W2, r1(b2))
        else:
            hf, ah, bd, eh = _nodeup_call(
                hn, hf, ps, r1(bnh_g[l]), r1(bnh_b[l]),
                Aw[l + 1], r1(Ab[l + 1]), Bw[l + 1], r1(Bb[l + 1]),
                Dw[l + 1], r1(Db[l + 1]), Ew[l + 1], r1(Eb[l + 1]))
            est3 = est.reshape(2, NS, 128)
            if l < NLAYER - 2:
                ef, ce2 = _edgeup_call(e2, ef, est3, r1(bne_g[l]),
                                       r1(bne_b[l]), Cw[l + 1], r1(Cb[l + 1]),
                                       True)
            else:
                (ce2,) = _edgeup_call(e2, ef, est3, r1(bne_g[l]),
                                      r1(bne_b[l]), Cw[l + 1], r1(Cb[l + 1]),
                                      False)
    return y


# speedup vs baseline: 1.8146x; 1.0177x over previous
import functools
import jax
import jax.numpy as jnp
import numpy as np
from jax import lax
from jax.experimental import pallas as pl
from jax.experimental.pallas import tpu as pltpu
from jax.experimental.pallas import tpu_sc as plsc

NN = 10000
NEDGE = 320000
DIM = 128
NLAYER = 4
NC = 2
NS = 16
EC = 40
EPT = NEDGE // NS
NCHUNK = EPT // EC
NHALF = NCHUNK // 2
ROWS_PT = NN // NS
BN_EPS = 1e-5
DEN_EPS = 1e-6

BN_NODE = 1000
BN_GRID = NN // BN_NODE
BE_EDGE = 4000
BE_GRID = NEDGE // BE_EDGE


def _sc_edge_body(last, icat_h, bdtab_h, ehtab_h, ce_h,
                  *refs):
    if last:
        (nd_h,
         idxv0, idxv1, bdv0, bdv1, ehv0, ehv1, cev0, cev1,
         env0, env1, statv, zbuf, acc,
         sbd0, sbd1, seh0, seh1, sce0, sce1, sen0, sen1) = refs
        e2_h = None
        stats_h = None
    else:
        (e2_h, nd_h, stats_h,
         idxv0, idxv1, bdv0, bdv1, ehv0, ehv1, cev0, cev1,
         env0, env1, statv, zbuf, acc,
         sbd0, sbd1, seh0, seh1, sce0, sce1, sen0, sen1) = refs
    c = lax.axis_index("c")
    s = lax.axis_index("s")

    zero = jnp.zeros((16,), jnp.float32)

    def _zrow(i, carry):
        for q in range(8):
            zbuf[i, pl.ds(q * 16, 16)] = zero
        return carry
    lax.fori_loop(0, 40, _zrow, 0)

    @pl.when(s < 10)
    def _zero_acc():
        for kk in range(25):
            pltpu.sync_copy(zbuf, acc.at[pl.ds(s * 1000 + kk * 40, 40)])
    plsc.subcore_barrier()

    def _load_idx(k, idxv):
        pltpu.sync_copy(icat_h.at[c * (NS * NCHUNK) + s * NCHUNK + k], idxv)

    def _issue(k, idxv, bdv, ehv, cev, sbd, seh, sce):
        base = s * EPT + k * EC
        pltpu.async_copy(bdtab_h.at[idxv.at[0]], bdv, sbd)
        pltpu.async_copy(ehtab_h.at[idxv.at[1]], ehv, seh)
        pltpu.async_copy(ce_h.at[c, pl.ds(base, EC)], cev, sce)

    def _drain(bdv, ehv, cev, sbd, seh, sce):
        pltpu.make_async_copy(bdtab_h.at[idxv0.at[0]], bdv, sbd).wait()
        pltpu.make_async_copy(ehtab_h.at[idxv0.at[1]], ehv, seh).wait()
        pltpu.make_async_copy(ce_h.at[c, pl.ds(0, EC)], cev, sce).wait()

    def _compute(k, idxv, bdv, ehv, cev, env, cr):
        def _erow(j, cr2):
            cr2 = list(cr2)
            for q in range(4):
                dq = bdv[j, pl.ds(64 + q * 16, 16)]
                ehq = ehv[j, pl.ds(c * 64 + q * 16, 16)]
                ceq = cev[j, pl.ds(q * 16, 16)]
                ev = ceq + dq + ehq
                if not last:
                    env[j, pl.ds(q * 16, 16)] = ev
                    cr2[q] = cr2[q] + ev
                    cr2[4 + q] = cr2[4 + q] + ev * ev
                sig = 1.0 / (1.0 + jnp.exp(-ev))
                bq = bdv[j, pl.ds(q * 16, 16)]
                bdv[j, pl.ds(q * 16, 16)] = sig * bq
                bdv[j, pl.ds(64 + q * 16, 16)] = sig
            return tuple(cr2)

        cr = lax.fori_loop(0, EC, _erow, cr, unroll=4)
        pltpu.sync_copy(bdv, acc.at[idxv.at[1]], add=True)
        return cr

    _load_idx(0, idxv0)
    _issue(0, idxv0, bdv0, ehv0, cev0, sbd0, seh0, sce0)

    def _env_issue(k, env, sen):
        base = s * EPT + k * EC
        pltpu.async_copy(env, e2_h.at[c, pl.ds(base, EC)], sen)

    def _env_drain(env, sen):
        pltpu.make_async_copy(env, e2_h.at[c, pl.ds(0, EC)], sen).wait()

    def _pair(kp, cr):
        ka = 2 * kp
        _load_idx(ka + 1, idxv1)
        _issue(ka + 1, idxv1, bdv1, ehv1, cev1, sbd1, seh1, sce1)
        _drain(bdv0, ehv0, cev0, sbd0, seh0, sce0)
        if not last:
            @pl.when(kp > 0)
            def _():
                _env_drain(env0, sen0)
        cr = _compute(ka, idxv0, bdv0, ehv0, cev0, env0, cr)
        if not last:
            _env_issue(ka, env0, sen0)

        @pl.when(kp + 1 < NHALF)
        def _prefetch_next():
            _load_idx(ka + 2, idxv0)
            _issue(ka + 2, idxv0, bdv0, ehv0, cev0, sbd0, seh0, sce0)
        _drain(bdv1, ehv1, cev1, sbd1, seh1, sce1)
        if not last:
            @pl.when(kp > 0)
            def _():
                _env_drain(env1, sen1)
        cr = _compute(ka + 1, idxv1, bdv1, ehv1, cev1, env1, cr)
        if not last:
            _env_issue(ka + 1, env1, sen1)
        return cr

    init = tuple(zero for _ in range(8))
    st = lax.fori_loop(0, NHALF, _pair, init)
    if not last:
        _env_drain(env0, sen0)
        _env_drain(env1, sen1)

    if not last:
        for q in range(4):
            statv[pl.ds(q * 16, 16)] = st[q]
            statv[pl.ds(64 + q * 16, 16)] = st[4 + q]
        pltpu.sync_copy(statv, stats_h.at[pl.ds((c * NS + s) * 128, 128)])

    plsc.subcore_barrier()

    @pl.when(s < 10)
    def _dump_acc():
        pltpu.sync_copy(acc.at[pl.ds(s * 1000, 1000)],
                        nd_h.at[c, pl.ds(s * 1000, 1000)])


def _sc_edge_pass(bdtab, ehtab, ce2, icat, last):
    mesh = plsc.VectorSubcoreMesh(core_axis_name="c", subcore_axis_name="s",
                                  num_cores=NC, num_subcores=NS)
    if last:
        out_type = (jax.ShapeDtypeStruct((NC, NN, 128), jnp.float32),)
    else:
        out_type = (
            jax.ShapeDtypeStruct((NC, NEDGE, 64), jnp.float32),
            jax.ShapeDtypeStruct((NC, NN, 128), jnp.float32),
            jax.ShapeDtypeStruct((NC * NS * 128,), jnp.float32),
        )
    scratch = [
        pltpu.VMEM((2, EC), jnp.int32),
        pltpu.VMEM((2, EC), jnp.int32),
        pltpu.VMEM((EC, 128), jnp.float32),
        pltpu.VMEM((EC, 128), jnp.float32),
        pltpu.VMEM((EC, 128), jnp.float32),
        pltpu.VMEM((EC, 128), jnp.float32),
        pltpu.VMEM((EC, 64), jnp.float32),
        pltpu.VMEM((EC, 64), jnp.float32),
        pltpu.VMEM((EC, 64), jnp.float32),
        pltpu.VMEM((EC, 64), jnp.float32),
        pltpu.VMEM((128,), jnp.float32),
        pltpu.VMEM((40, 128), jnp.float32),
        pltpu.VMEM_SHARED((NN, 128), jnp.float32),
    ] + [pltpu.SemaphoreType.DMA] * 8
    fn = pl.kernel(functools.partial(_sc_edge_body, last),
                   out_type=out_type, mesh=mesh, scratch_types=scratch)
    return fn(icat, bdtab, ehtab, ce2)


def _node_mats(hf, wa, ab, wb, bb, wd, db, we, eb,
               hf_out, ah_out, bd_out, eh_out):
    if hf_out is not None:
        hf_out[...] = hf
    f32 = jnp.float32
    Ah = jnp.dot(hf, wa[...], preferred_element_type=f32) + ab[...]
    Bh = jnp.dot(hf, wb[...], preferred_element_type=f32) + bb[...]
    Dh = jnp.dot(hf, wd[...], preferred_element_type=f32) + db[...]
    Eh = jnp.dot(hf, we[...], preferred_element_type=f32) + eb[...]
    ah_out[...] = Ah
    bd_out[0] = jnp.concatenate([Bh[:, :64], Dh[:, :64]], axis=1)
    bd_out[1] = jnp.concatenate([Bh[:, 64:], Dh[:, 64:]], axis=1)
    eh_out[...] = Eh


def _enc_node_body(h_ref, p_ref, acat, pw, pb2,
                   wa, ab, wb, bb, wd, db, we, eb,
                   hf_out, ah_out, bd_out, eh_out):
    f32 = jnp.float32
    hf = jnp.dot(p_ref[...], pw[...], preferred_element_type=f32) + pb2[...]
    iot = lax.broadcasted_iota(jnp.int32, (1, 128), 1)
    for i in range(9):
        oh = (h_ref[:, i:i + 1] == iot).astype(f32)
        hf = hf + jnp.dot(oh, acat[i], preferred_element_type=f32)
    _node_mats(hf, wa, ab, wb, bb, wd, db, we, eb,
               hf_out, ah_out, bd_out, eh_out)


def _enc_node_call(h, p, acat, pw, pb2, wa, ab, wb, bb, wd, db, we, eb):
    B = BN_NODE
    full = lambda *shape: pl.BlockSpec(shape, lambda i: (0,) * len(shape))
    blk = lambda *shape: pl.BlockSpec(shape, lambda i: (i,) + (0,) * (len(shape) - 1))
    blk2 = lambda *shape: pl.BlockSpec(shape, lambda i: (0, i) + (0,) * (len(shape) - 2))
    return pl.pallas_call(
        _enc_node_body,
        grid=(BN_GRID,),
        in_specs=[blk(B, 9), blk(B, 8), full(9, 128, 128), full(8, 128),
                  full(1, 128)] + [full(128, 128), full(1, 128)] * 4,
        out_specs=[blk(B, 128), blk(B, 128), blk2(2, B, 128), blk(B, 128)],
        out_shape=[
            jax.ShapeDtypeStruct((NN, 128), jnp.float32),
            jax.ShapeDtypeStruct((NN, 128), jnp.float32),
            jax.ShapeDtypeStruct((2, NN, 128), jnp.float32),
            jax.ShapeDtypeStruct((NN, 128), jnp.float32),
        ],
    )(h, p, acat, pw, pb2, wa, ab, wb, bb, wd, db, we, eb)


def _bond_edge_body(e_ref, bcat, cw, cb2, ef_out, ce_out):
    f32 = jnp.float32
    iot = lax.broadcasted_iota(jnp.int32, (1, 8), 1)
    ef = jnp.zeros((e_ref.shape[0], 128), f32)
    for i in range(3):
        oh = (e_ref[:, i:i + 1] == iot).astype(f32)
        ef = ef + jnp.dot(oh, bcat[i], preferred_element_type=f32)
    ef_out[...] = ef
    ce = jnp.dot(ef, cw[...], preferred_element_type=f32) + cb2[...]
    ce_out[0] = ce[:, :64]
    ce_out[1] = ce[:, 64:]


def _bond_edge_call(e, bcat, cw, cb2):
    B = BE_EDGE
    full = lambda *shape: pl.BlockSpec(shape, lambda i: (0,) * len(shape))
    blk = lambda *shape: pl.BlockSpec(shape, lambda i: (i,) + (0,) * (len(shape) - 1))
    blk2 = lambda *shape: pl.BlockSpec(shape, lambda i: (0, i) + (0,) * (len(shape) - 2))
    return pl.pallas_call(
        _bond_edge_body,
        grid=(BE_GRID,),
        in_specs=[blk(B, 3), full(3, 8, 128), full(128, 128), full(1, 128)],
        out_specs=[blk(B, 128), blk2(2, B, 64)],
        out_shape=[
            jax.ShapeDtypeStruct((NEDGE, 128), jnp.float32),
            jax.ShapeDtypeStruct((2, NEDGE, 64), jnp.float32),
        ],
    )(e, bcat, cw, cb2)


def _hnew_body(nd_ref, ah_ref, hn_out, ps_out):
    num = jnp.concatenate([nd_ref[0][:, :64], nd_ref[1][:, :64]], axis=1)
    den = jnp.concatenate([nd_ref[0][:, 64:], nd_ref[1][:, 64:]], axis=1)
    hn = ah_ref[...] + num / (den + DEN_EPS)
    hn_out[...] = hn
    s1 = jnp.sum(hn, axis=0, keepdims=True)
    s2 = jnp.sum(hn * hn, axis=0, keepdims=True)
    ps_out[...] = jnp.concatenate(
        [s1, s2, jnp.zeros((6, 128), jnp.float32)], axis=0).reshape(1, 8, 128)


def _hnew_call(nd, ah):
    B = BN_NODE
    blk = lambda *shape: pl.BlockSpec(shape, lambda i: (i,) + (0,) * (len(shape) - 1))
    blk2 = lambda *shape: pl.BlockSpec(shape, lambda i: (0, i) + (0,) * (len(shape) - 2))
    return pl.pallas_call(
        _hnew_body,
        grid=(BN_GRID,),
        in_specs=[blk2(2, B, 128), blk(B, 128)],
        out_specs=[blk(B, 128), blk(1, 8, 128)],
        out_shape=[
            jax.ShapeDtypeStruct((NN, 128), jnp.float32),
            jax.ShapeDtypeStruct((BN_GRID, 8, 128), jnp.float32),
        ],
    )(nd, ah)


def _node_stats(ps_ref):
    m = jnp.sum(ps_ref[:, 0, :], axis=0, keepdims=True) / NN
    v = jnp.sum(ps_ref[:, 1, :], axis=0, keepdims=True) / NN - m * m
    return m, v


def _nodeup_body(hn_ref, hfp_ref, ps_ref, g2, b2,
                 wa, ab, wb, bb, wd, db, we, eb,
                 hf_out, ah_out, bd_out, eh_out):
    m, v = _node_stats(ps_ref)
    bn = g2[...] * (hn_ref[...] - m) * lax.rsqrt(v + BN_EPS) + b2[...]
    hf = hfp_ref[...] + jnp.maximum(bn, 0.0)
    _node_mats(hf, wa, ab, wb, bb, wd, db, we, eb,
               hf_out, ah_out, bd_out, eh_out)


def _nodeup_call(hn, hfp, ps, g2, b2, wa, ab, wb, bb, wd, db, we, eb):
    B = BN_NODE
    full = lambda *shape: pl.BlockSpec(shape, lambda i: (0,) * len(shape))
    blk = lambda *shape: pl.BlockSpec(shape, lambda i: (i,) + (0,) * (len(shape) - 1))
    blk2 = lambda *shape: pl.BlockSpec(shape, lambda i: (0, i) + (0,) * (len(shape) - 2))
    return pl.pallas_call(
        _nodeup_body,
        grid=(BN_GRID,),
        in_specs=[blk(B, 128), blk(B, 128), full(BN_GRID, 8, 128),
                  full(1, 128), full(1, 128)]
                 + [full(128, 128), full(1, 128)] * 4,
        out_specs=[blk(B, 128), blk(B, 128), blk2(2, B, 128), blk(B, 128)],
        out_shape=[
            jax.ShapeDtypeStruct((NN, 128), jnp.float32),
            jax.ShapeDtypeStruct((NN, 128), jnp.float32),
            jax.ShapeDtypeStruct((2, NN, 128), jnp.float32),
            jax.ShapeDtypeStruct((NN, 128), jnp.float32),
        ],
    )(hn, hfp, ps, g2, b2, wa, ab, wb, bb, wd, db, we, eb)


def _edge_stats(est_ref):
    e0 = jnp.sum(est_ref[0], axis=0)
    e1 = jnp.sum(est_ref[1], axis=0)
    m = jnp.concatenate([e0[:64], e1[:64]]).reshape(1, 128) / NEDGE
    q = jnp.concatenate([e0[64:], e1[64:]]).reshape(1, 128) / NEDGE
    return m, q - m * m


def _edgeup_body(write_ef, e2_ref, ef_ref, est_ref, g2, b2, cw, cb2, *outs):
    if write_ef:
        ef_out, ce_out = outs
    else:
        (ce_out,) = outs
    m, v = _edge_stats(est_ref)
    en = jnp.concatenate([e2_ref[0], e2_ref[1]], axis=1)
    bn = g2[...] * (en - m) * lax.rsqrt(v + BN_EPS) + b2[...]
    ef = ef_ref[...] + jnp.maximum(bn, 0.0)
    if write_ef:
        ef_out[...] = ef
    ce = jnp.dot(ef, cw[...], preferred_element_type=jnp.float32) + cb2[...]
    ce_out[0] = ce[:, :64]
    ce_out[1] = ce[:, 64:]


def _edgeup_call(e2, ef, est, g2, b2, cw, cb2, write_ef):
    B = BE_EDGE
    full = lambda *shape: pl.BlockSpec(shape, lambda i: (0,) * len(shape))
    blk = lambda *shape: pl.BlockSpec(shape, lambda i: (i,) + (0,) * (len(shape) - 1))
    blk2 = lambda *shape: pl.BlockSpec(shape, lambda i: (0, i) + (0,) * (len(shape) - 2))
    out_specs = [blk2(2, B, 64)]
    out_shape = [jax.ShapeDtypeStruct((2, NEDGE, 64), jnp.float32)]
    if write_ef:
        out_specs = [blk(B, 128)] + out_specs
        out_shape = [jax.ShapeDtypeStruct((NEDGE, 128), jnp.float32)] + out_shape
    return pl.pallas_call(
        functools.partial(_edgeup_body, write_ef),
        grid=(BE_GRID,),
        in_specs=[blk2(2, B, 64), blk(B, 128), full(2, NS, 128),
                  full(1, 128), full(1, 128), full(128, 128), full(1, 128)],
        out_specs=out_specs,
        out_shape=out_shape,
    )(e2, ef, est, g2, b2, cw, cb2)


def _final_body(hn_ref, hfp_ref, ps_ref, g2, b2,
                w0, b0r, w1, b1r, w2, b2r, y_out):
    m, v = _node_stats(ps_ref)
    bn = g2[...] * (hn_ref[...] - m) * lax.rsqrt(v + BN_EPS) + b2[...]
    hf = hfp_ref[...] + jnp.maximum(bn, 0.0)
    hg = jnp.sum(hf, axis=0, keepdims=True) / NN
    f32 = jnp.float32
    y = jnp.maximum(jnp.dot(hg, w0[...], preferred_element_type=f32) + b0r[...], 0.0)
    y = jnp.maximum(jnp.dot(y, w1[...], preferred_element_type=f32) + b1r[...], 0.0)
    y = jnp.dot(y, w2[...], preferred_element_type=f32) + b2r[...]
    y_out[...] = y


def _final_call(hn, hfp, ps, g2, b2, w0, b0r, w1, b1r, w2, b2r):
    full = lambda *shape: pl.BlockSpec(shape, lambda i: (0,) * len(shape))
    return pl.pallas_call(
        _final_body,
        grid=(1,),
        in_specs=[full(NN, 128), full(NN, 128), full(BN_GRID, 8, 128),
                  full(1, 128), full(1, 128), full(128, 64), full(1, 64),
                  full(64, 32), full(1, 32), full(32, 128), full(1, 128)],
        out_specs=[full(1, 128)],
        out_shape=[jax.ShapeDtypeStruct((1, 128), jnp.float32)],
    )(hn, hfp, ps, g2, b2, w0, b0r, w1, b1r, w2, b2r)


def kernel(h, p, e, snorm_n, edge_index,
           atom_emb_0, atom_emb_1, atom_emb_2, atom_emb_3, atom_emb_4,
           atom_emb_5, atom_emb_6, atom_emb_7, atom_emb_8,
           bond_emb_0, bond_emb_1, bond_emb_2,
           Pw, Pb, Aw, Ab, Bw, Bb, Cw, Cb, Dw, Db, Ew, Eb,
           bnh_g, bnh_b, bne_g, bne_b, W0, b0, W1, b1, W2, b2):
    f32 = jnp.float32
    atom_tabs = [atom_emb_0, atom_emb_1, atom_emb_2, atom_emb_3, atom_emb_4,
                 atom_emb_5, atom_emb_6, atom_emb_7, atom_emb_8]
    acat = jnp.stack([jnp.pad(t, ((0, 128 - t.shape[0]), (0, 0)))
                      for t in atom_tabs])
    bond_tabs = [bond_emb_0, bond_emb_1, bond_emb_2]
    bcat = jnp.stack([jnp.pad(t, ((0, 8 - t.shape[0]), (0, 0)))
                      for t in bond_tabs])
    r1 = lambda x: x.reshape(1, -1).astype(f32)

    src = edge_index[0].astype(jnp.int32)
    dst = edge_index[1].astype(jnp.int32)
    src2 = jnp.concatenate([src, src + NN])
    src2r = src2.reshape(2, NS, NCHUNK, EC)
    dstr = jnp.broadcast_to(dst.reshape(1, NS, NCHUNK, EC),
                            (2, NS, NCHUNK, EC))
    icat = jnp.stack([src2r, dstr], axis=3).reshape(2 * NS * NCHUNK, 2, EC)

    h = h.astype(jnp.int32)
    e = e.astype(jnp.int32)

    hf, ah, bd, eh = _enc_node_call(
        h, p.astype(f32), acat, Pw, r1(Pb),
        Aw[0], r1(Ab[0]), Bw[0], r1(Bb[0]), Dw[0], r1(Db[0]), Ew[0], r1(Eb[0]))
    ef, ce2 = _bond_edge_call(e, bcat, Cw[0], r1(Cb[0]))

    for l in range(NLAYER):
        last = (l == NLAYER - 1)
        bdtab = bd.reshape(2 * NN, 128)
        if last:
            (nd,) = _sc_edge_pass(bdtab, eh, ce2, icat, True)
        else:
            e2, nd, est = _sc_edge_pass(bdtab, eh, ce2, icat, False)
        hn, ps = _hnew_call(nd, ah)
        if last:
            (y,) = _final_call(hn, hf, ps, r1(bnh_g[l]), r1(bnh_b[l]),
                               W0, r1(b0), W1, r1(b1), W2, r1(b2))
        else:
            hf, ah, bd, eh = _nodeup_call(
                hn, hf, ps, r1(bnh_g[l]), r1(bnh_b[l]),
                Aw[l + 1], r1(Ab[l + 1]), Bw[l + 1], r1(Bb[l + 1]),
                Dw[l + 1], r1(Db[l + 1]), Ew[l + 1], r1(Eb[l + 1]))
            est3 = est.reshape(2, NS, 128)
            if l < NLAYER - 2:
                ef, ce2 = _edgeup_call(e2, ef, est3, r1(bne_g[l]),
                                       r1(bne_b[l]), Cw[l + 1], r1(Cb[l + 1]),
                                       True)
            else:
                (ce2,) = _edgeup_call(e2, ef, est3, r1(bne_g[l]),
                                      r1(bne_b[l]), Cw[l + 1], r1(Cb[l + 1]),
                                      False)
    return y


# speedup vs baseline: 2.1635x; 1.1923x over previous
import functools
import jax
import jax.numpy as jnp
import numpy as np
from jax import lax
from jax.experimental import pallas as pl
from jax.experimental.pallas import tpu as pltpu
from jax.experimental.pallas import tpu_sc as plsc

NN = 10000
NEDGE = 320000
DIM = 128
NLAYER = 4
NC = 2
NS = 16
EC = 40
EPT = NEDGE // NS
NCHUNK = EPT // EC
NHALF = NCHUNK // 2
ROWS_PT = NN // NS
BN_EPS = 1e-5
DEN_EPS = 1e-6

BN_NODE = 1000
BN_GRID = NN // BN_NODE
BE_EDGE = 4000
BE_GRID = NEDGE // BE_EDGE


def _sc_edge_body(last, icat_h, bdtab_h, ehtab_h, ce_h,
                  *refs):
    if last:
        (nd_h,
         idxv0, idxv1, bdv0, bdv1, ehv0, ehv1, cev0, cev1,
         env0, env1, statv, zbuf, acc,
         sbd0, sbd1, seh0, seh1, sce0, sce1, sen0, sen1) = refs
        e2_h = None
        stats_h = None
    else:
        (e2_h, nd_h, stats_h,
         idxv0, idxv1, bdv0, bdv1, ehv0, ehv1, cev0, cev1,
         env0, env1, statv, zbuf, acc,
         sbd0, sbd1, seh0, seh1, sce0, sce1, sen0, sen1) = refs
    c = lax.axis_index("c")
    s = lax.axis_index("s")

    zero = jnp.zeros((16,), jnp.float32)

    def _zrow(i, carry):
        for q in range(8):
            zbuf[i, pl.ds(q * 16, 16)] = zero
        return carry
    lax.fori_loop(0, 40, _zrow, 0)

    @pl.when(s < 10)
    def _zero_acc():
        for kk in range(25):
            pltpu.sync_copy(zbuf, acc.at[pl.ds(s * 1000 + kk * 40, 40)])
    plsc.subcore_barrier()

    def _load_idx(k, idxv):
        pltpu.sync_copy(icat_h.at[c * (NS * NCHUNK) + s * NCHUNK + k], idxv)

    def _issue(k, idxv, bdv, ehv, cev, sbd, seh, sce):
        base = s * EPT + k * EC
        pltpu.async_copy(bdtab_h.at[idxv.at[0]], bdv, sbd)
        pltpu.async_copy(ehtab_h.at[idxv.at[1]], ehv, seh)
        pltpu.async_copy(ce_h.at[c, pl.ds(base, EC)], cev, sce)

    def _drain(bdv, ehv, cev, sbd, seh, sce):
        pltpu.make_async_copy(bdtab_h.at[idxv0.at[0]], bdv, sbd).wait()
        pltpu.make_async_copy(ehtab_h.at[idxv0.at[1]], ehv, seh).wait()
        pltpu.make_async_copy(ce_h.at[c, pl.ds(0, EC)], cev, sce).wait()

    def _compute(k, idxv, bdv, ehv, cev, env, cr):
        def _erow(j, cr2):
            cr2 = list(cr2)
            for q in range(4):
                dq = bdv[j, pl.ds(64 + q * 16, 16)]
                ehq = ehv[j, pl.ds(c * 64 + q * 16, 16)]
                ceq = cev[j, pl.ds(q * 16, 16)]
                ev = ceq + dq + ehq
                if not last:
                    env[j, pl.ds(q * 16, 16)] = ev
                    cr2[q] = cr2[q] + ev
                    cr2[4 + q] = cr2[4 + q] + ev * ev
                sig = 1.0 / (1.0 + jnp.exp(-ev))
                bq = bdv[j, pl.ds(q * 16, 16)]
                bdv[j, pl.ds(q * 16, 16)] = sig * bq
                bdv[j, pl.ds(64 + q * 16, 16)] = sig
            return tuple(cr2)

        cr = plsc.parallel_loop(0, EC, carry=cr, unroll=4)(_erow)
        pltpu.sync_copy(bdv, acc.at[idxv.at[1]], add=True)
        return cr

    _load_idx(0, idxv0)
    _issue(0, idxv0, bdv0, ehv0, cev0, sbd0, seh0, sce0)

    def _env_issue(k, env, sen):
        base = s * EPT + k * EC
        pltpu.async_copy(env, e2_h.at[c, pl.ds(base, EC)], sen)

    def _env_drain(env, sen):
        pltpu.make_async_copy(env, e2_h.at[c, pl.ds(0, EC)], sen).wait()

    def _pair(kp, cr):
        ka = 2 * kp
        _load_idx(ka + 1, idxv1)
        _issue(ka + 1, idxv1, bdv1, ehv1, cev1, sbd1, seh1, sce1)
        _drain(bdv0, ehv0, cev0, sbd0, seh0, sce0)
        if not last:
            @pl.when(kp > 0)
            def _():
                _env_drain(env0, sen0)
        cr = _compute(ka, idxv0, bdv0, ehv0, cev0, env0, cr)
        if not last:
            _env_issue(ka, env0, sen0)

        @pl.when(kp + 1 < NHALF)
        def _prefetch_next():
            _load_idx(ka + 2, idxv0)
            _issue(ka + 2, idxv0, bdv0, ehv0, cev0, sbd0, seh0, sce0)
        _drain(bdv1, ehv1, cev1, sbd1, seh1, sce1)
        if not last:
            @pl.when(kp > 0)
            def _():
                _env_drain(env1, sen1)
        cr = _compute(ka + 1, idxv1, bdv1, ehv1, cev1, env1, cr)
        if not last:
            _env_issue(ka + 1, env1, sen1)
        return cr

    init = tuple(zero for _ in range(8))
    st = lax.fori_loop(0, NHALF, _pair, init)
    if not last:
        _env_drain(env0, sen0)
        _env_drain(env1, sen1)

    if not last:
        for q in range(4):
            statv[pl.ds(q * 16, 16)] = st[q]
            statv[pl.ds(64 + q * 16, 16)] = st[4 + q]
        pltpu.sync_copy(statv, stats_h.at[pl.ds((c * NS + s) * 128, 128)])

    plsc.subcore_barrier()

    @pl.when(s < 10)
    def _dump_acc():
        pltpu.sync_copy(acc.at[pl.ds(s * 1000, 1000)],
                        nd_h.at[c, pl.ds(s * 1000, 1000)])


def _sc_edge_pass(bdtab, ehtab, ce2, icat, last):
    mesh = plsc.VectorSubcoreMesh(core_axis_name="c", subcore_axis_name="s",
                                  num_cores=NC, num_subcores=NS)
    if last:
        out_type = (jax.ShapeDtypeStruct((NC, NN, 128), jnp.float32),)
    else:
        out_type = (
            jax.ShapeDtypeStruct((NC, NEDGE, 64), jnp.float32),
            jax.ShapeDtypeStruct((NC, NN, 128), jnp.float32),
            jax.ShapeDtypeStruct((NC * NS * 128,), jnp.float32),
        )
    scratch = [
        pltpu.VMEM((2, EC), jnp.int32),
        pltpu.VMEM((2, EC), jnp.int32),
        pltpu.VMEM((EC, 128), jnp.float32),
        pltpu.VMEM((EC, 128), jnp.float32),
        pltpu.VMEM((EC, 128), jnp.float32),
        pltpu.VMEM((EC, 128), jnp.float32),
        pltpu.VMEM((EC, 64), jnp.float32),
        pltpu.VMEM((EC, 64), jnp.float32),
        pltpu.VMEM((EC, 64), jnp.float32),
        pltpu.VMEM((EC, 64), jnp.float32),
        pltpu.VMEM((128,), jnp.float32),
        pltpu.VMEM((40, 128), jnp.float32),
        pltpu.VMEM_SHARED((NN, 128), jnp.float32),
    ] + [pltpu.SemaphoreType.DMA] * 8
    fn = pl.kernel(functools.partial(_sc_edge_body, last),
                   out_type=out_type, mesh=mesh, scratch_types=scratch)
    return fn(icat, bdtab, ehtab, ce2)


def _node_mats(hf, wa, ab, wb, bb, wd, db, we, eb,
               hf_out, ah_out, bd_out, eh_out):
    if hf_out is not None:
        hf_out[...] = hf
    f32 = jnp.float32
    Ah = jnp.dot(hf, wa[...], preferred_element_type=f32) + ab[...]
    Bh = jnp.dot(hf, wb[...], preferred_element_type=f32) + bb[...]
    Dh = jnp.dot(hf, wd[...], preferred_element_type=f32) + db[...]
    Eh = jnp.dot(hf, we[...], preferred_element_type=f32) + eb[...]
    ah_out[...] = Ah
    bd_out[0] = jnp.concatenate([Bh[:, :64], Dh[:, :64]], axis=1)
    bd_out[1] = jnp.concatenate([Bh[:, 64:], Dh[:, 64:]], axis=1)
    eh_out[...] = Eh


def _enc_node_body(h_ref, p_ref, acat, pw, pb2,
                   wa, ab, wb, bb, wd, db, we, eb,
                   hf_out, ah_out, bd_out, eh_out):
    f32 = jnp.float32
    hf = jnp.dot(p_ref[...], pw[...], preferred_element_type=f32) + pb2[...]
    iot = lax.broadcasted_iota(jnp.int32, (1, 128), 1)
    for i in range(9):
        oh = (h_ref[:, i:i + 1] == iot).astype(f32)
        hf = hf + jnp.dot(oh, acat[i], preferred_element_type=f32)
    _node_mats(hf, wa, ab, wb, bb, wd, db, we, eb,
               hf_out, ah_out, bd_out, eh_out)


def _enc_node_call(h, p, acat, pw, pb2, wa, ab, wb, bb, wd, db, we, eb):
    B = BN_NODE
    full = lambda *shape: pl.BlockSpec(shape, lambda i: (0,) * len(shape))
    blk = lambda *shape: pl.BlockSpec(shape, lambda i: (i,) + (0,) * (len(shape) - 1))
    blk2 = lambda *shape: pl.BlockSpec(shape, lambda i: (0, i) + (0,) * (len(shape) - 2))
    return pl.pallas_call(
        _enc_node_body,
        grid=(BN_GRID,),
        in_specs=[blk(B, 9), blk(B, 8), full(9, 128, 128), full(8, 128),
                  full(1, 128)] + [full(128, 128), full(1, 128)] * 4,
        out_specs=[blk(B, 128), blk(B, 128), blk2(2, B, 128), blk(B, 128)],
        out_shape=[
            jax.ShapeDtypeStruct((NN, 128), jnp.float32),
            jax.ShapeDtypeStruct((NN, 128), jnp.float32),
            jax.ShapeDtypeStruct((2, NN, 128), jnp.float32),
            jax.ShapeDtypeStruct((NN, 128), jnp.float32),
        ],
    )(h, p, acat, pw, pb2, wa, ab, wb, bb, wd, db, we, eb)


def _bond_edge_body(e_ref, bcat, cw, cb2, ef_out, ce_out):
    f32 = jnp.float32
    iot = lax.broadcasted_iota(jnp.int32, (1, 8), 1)
    ef = jnp.zeros((e_ref.shape[0], 128), f32)
    for i in range(3):
        oh = (e_ref[:, i:i + 1] == iot).astype(f32)
        ef = ef + jnp.dot(oh, bcat[i], preferred_element_type=f32)
    ef_out[...] = ef
    ce = jnp.dot(ef, cw[...], preferred_element_type=f32) + cb2[...]
    ce_out[0] = ce[:, :64]
    ce_out[1] = ce[:, 64:]


def _bond_edge_call(e, bcat, cw, cb2):
    B = BE_EDGE
    full = lambda *shape: pl.BlockSpec(shape, lambda i: (0,) * len(shape))
    blk = lambda *shape: pl.BlockSpec(shape, lambda i: (i,) + (0,) * (len(shape) - 1))
    blk2 = lambda *shape: pl.BlockSpec(shape, lambda i: (0, i) + (0,) * (len(shape) - 2))
    return pl.pallas_call(
        _bond_edge_body,
        grid=(BE_GRID,),
        in_specs=[blk(B, 3), full(3, 8, 128), full(128, 128), full(1, 128)],
        out_specs=[blk(B, 128), blk2(2, B, 64)],
        out_shape=[
            jax.ShapeDtypeStruct((NEDGE, 128), jnp.float32),
            jax.ShapeDtypeStruct((2, NEDGE, 64), jnp.float32),
        ],
    )(e, bcat, cw, cb2)


def _hnew_body(nd_ref, ah_ref, hn_out, ps_out):
    num = jnp.concatenate([nd_ref[0][:, :64], nd_ref[1][:, :64]], axis=1)
    den = jnp.concatenate([nd_ref[0][:, 64:], nd_ref[1][:, 64:]], axis=1)
    hn = ah_ref[...] + num / (den + DEN_EPS)
    hn_out[...] = hn
    s1 = jnp.sum(hn, axis=0, keepdims=True)
    s2 = jnp.sum(hn * hn, axis=0, keepdims=True)
    ps_out[...] = jnp.concatenate(
        [s1, s2, jnp.zeros((6, 128), jnp.float32)], axis=0).reshape(1, 8, 128)


def _hnew_call(nd, ah):
    B = BN_NODE
    blk = lambda *shape: pl.BlockSpec(shape, lambda i: (i,) + (0,) * (len(shape) - 1))
    blk2 = lambda *shape: pl.BlockSpec(shape, lambda i: (0, i) + (0,) * (len(shape) - 2))
    return pl.pallas_call(
        _hnew_body,
        grid=(BN_GRID,),
        in_specs=[blk2(2, B, 128), blk(B, 128)],
        out_specs=[blk(B, 128), blk(1, 8, 128)],
        out_shape=[
            jax.ShapeDtypeStruct((NN, 128), jnp.float32),
            jax.ShapeDtypeStruct((BN_GRID, 8, 128), jnp.float32),
        ],
    )(nd, ah)


def _node_stats(ps_ref):
    m = jnp.sum(ps_ref[:, 0, :], axis=0, keepdims=True) / NN
    v = jnp.sum(ps_ref[:, 1, :], axis=0, keepdims=True) / NN - m * m
    return m, v


def _nodeup_body(hn_ref, hfp_ref, ps_ref, g2, b2,
                 wa, ab, wb, bb, wd, db, we, eb,
                 hf_out, ah_out, bd_out, eh_out):
    m, v = _node_stats(ps_ref)
    bn = g2[...] * (hn_ref[...] - m) * lax.rsqrt(v + BN_EPS) + b2[...]
    hf = hfp_ref[...] + jnp.maximum(bn, 0.0)
    _node_mats(hf, wa, ab, wb, bb, wd, db, we, eb,
               hf_out, ah_out, bd_out, eh_out)


def _nodeup_call(hn, hfp, ps, g2, b2, wa, ab, wb, bb, wd, db, we, eb):
    B = BN_NODE
    full = lambda *shape: pl.BlockSpec(shape, lambda i: (0,) * len(shape))
    blk = lambda *shape: pl.BlockSpec(shape, lambda i: (i,) + (0,) * (len(shape) - 1))
    blk2 = lambda *shape: pl.BlockSpec(shape, lambda i: (0, i) + (0,) * (len(shape) - 2))
    return pl.pallas_call(
        _nodeup_body,
        grid=(BN_GRID,),
        in_specs=[blk(B, 128), blk(B, 128), full(BN_GRID, 8, 128),
                  full(1, 128), full(1, 128)]
                 + [full(128, 128), full(1, 128)] * 4,
        out_specs=[blk(B, 128), blk(B, 128), blk2(2, B, 128), blk(B, 128)],
        out_shape=[
            jax.ShapeDtypeStruct((NN, 128), jnp.float32),
            jax.ShapeDtypeStruct((NN, 128), jnp.float32),
            jax.ShapeDtypeStruct((2, NN, 128), jnp.float32),
            jax.ShapeDtypeStruct((NN, 128), jnp.float32),
        ],
    )(hn, hfp, ps, g2, b2, wa, ab, wb, bb, wd, db, we, eb)


def _edge_stats(est_ref):
    e0 = jnp.sum(est_ref[0], axis=0)
    e1 = jnp.sum(est_ref[1], axis=0)
    m = jnp.concatenate([e0[:64], e1[:64]]).reshape(1, 128) / NEDGE
    q = jnp.concatenate([e0[64:], e1[64:]]).reshape(1, 128) / NEDGE
    return m, q - m * m


def _edgeup_body(write_ef, e2_ref, ef_ref, est_ref, g2, b2, cw, cb2, *outs):
    if write_ef:
        ef_out, ce_out = outs
    else:
        (ce_out,) = outs
    m, v = _edge_stats(est_ref)
    en = jnp.concatenate([e2_ref[0], e2_ref[1]], axis=1)
    bn = g2[...] * (en - m) * lax.rsqrt(v + BN_EPS) + b2[...]
    ef = ef_ref[...] + jnp.maximum(bn, 0.0)
    if write_ef:
        ef_out[...] = ef
    ce = jnp.dot(ef, cw[...], preferred_element_type=jnp.float32) + cb2[...]
    ce_out[0] = ce[:, :64]
    ce_out[1] = ce[:, 64:]


def _edgeup_call(e2, ef, est, g2, b2, cw, cb2, write_ef):
    B = BE_EDGE
    full = lambda *shape: pl.BlockSpec(shape, lambda i: (0,) * len(shape))
    blk = lambda *shape: pl.BlockSpec(shape, lambda i: (i,) + (0,) * (len(shape) - 1))
    blk2 = lambda *shape: pl.BlockSpec(shape, lambda i: (0, i) + (0,) * (len(shape) - 2))
    out_specs = [blk2(2, B, 64)]
    out_shape = [jax.ShapeDtypeStruct((2, NEDGE, 64), jnp.float32)]
    if write_ef:
        out_specs = [blk(B, 128)] + out_specs
        out_shape = [jax.ShapeDtypeStruct((NEDGE, 128), jnp.float32)] + out_shape
    return pl.pallas_call(
        functools.partial(_edgeup_body, write_ef),
        grid=(BE_GRID,),
        in_specs=[blk2(2, B, 64), blk(B, 128), full(2, NS, 128),
                  full(1, 128), full(1, 128), full(128, 128), full(1, 128)],
        out_specs=out_specs,
        out_shape=out_shape,
    )(e2, ef, est, g2, b2, cw, cb2)


def _final_body(hn_ref, hfp_ref, ps_ref, g2, b2,
                w0, b0r, w1, b1r, w2, b2r, y_out):
    m, v = _node_stats(ps_ref)
    bn = g2[...] * (hn_ref[...] - m) * lax.rsqrt(v + BN_EPS) + b2[...]
    hf = hfp_ref[...] + jnp.maximum(bn, 0.0)
    hg = jnp.sum(hf, axis=0, keepdims=True) / NN
    f32 = jnp.float32
    y = jnp.maximum(jnp.dot(hg, w0[...], preferred_element_type=f32) + b0r[...], 0.0)
    y = jnp.maximum(jnp.dot(y, w1[...], preferred_element_type=f32) + b1r[...], 0.0)
    y = jnp.dot(y, w2[...], preferred_element_type=f32) + b2r[...]
    y_out[...] = y


def _final_call(hn, hfp, ps, g2, b2, w0, b0r, w1, b1r, w2, b2r):
    full = lambda *shape: pl.BlockSpec(shape, lambda i: (0,) * len(shape))
    return pl.pallas_call(
        _final_body,
        grid=(1,),
        in_specs=[full(NN, 128), full(NN, 128), full(BN_GRID, 8, 128),
                  full(1, 128), full(1, 128), full(128, 64), full(1, 64),
                  full(64, 32), full(1, 32), full(32, 128), full(1, 128)],
        out_specs=[full(1, 128)],
        out_shape=[jax.ShapeDtypeStruct((1, 128), jnp.float32)],
    )(hn, hfp, ps, g2, b2, w0, b0r, w1, b1r, w2, b2r)


def kernel(h, p, e, snorm_n, edge_index,
           atom_emb_0, atom_emb_1, atom_emb_2, atom_emb_3, atom_emb_4,
           atom_emb_5, atom_emb_6, atom_emb_7, atom_emb_8,
           bond_emb_0, bond_emb_1, bond_emb_2,
           Pw, Pb, Aw, Ab, Bw, Bb, Cw, Cb, Dw, Db, Ew, Eb,
           bnh_g, bnh_b, bne_g, bne_b, W0, b0, W1, b1, W2, b2):
    f32 = jnp.float32
    atom_tabs = [atom_emb_0, atom_emb_1, atom_emb_2, atom_emb_3, atom_emb_4,
                 atom_emb_5, atom_emb_6, atom_emb_7, atom_emb_8]
    acat = jnp.stack([jnp.pad(t, ((0, 128 - t.shape[0]), (0, 0)))
                      for t in atom_tabs])
    bond_tabs = [bond_emb_0, bond_emb_1, bond_emb_2]
    bcat = jnp.stack([jnp.pad(t, ((0, 8 - t.shape[0]), (0, 0)))
                      for t in bond_tabs])
    r1 = lambda x: x.reshape(1, -1).astype(f32)

    src = edge_index[0].astype(jnp.int32)
    dst = edge_index[1].astype(jnp.int32)
    src2 = jnp.concatenate([src, src + NN])
    src2r = src2.reshape(2, NS, NCHUNK, EC)
    dstr = jnp.broadcast_to(dst.reshape(1, NS, NCHUNK, EC),
                            (2, NS, NCHUNK, EC))
    icat = jnp.stack([src2r, dstr], axis=3).reshape(2 * NS * NCHUNK, 2, EC)

    h = h.astype(jnp.int32)
    e = e.astype(jnp.int32)

    hf, ah, bd, eh = _enc_node_call(
        h, p.astype(f32), acat, Pw, r1(Pb),
        Aw[0], r1(Ab[0]), Bw[0], r1(Bb[0]), Dw[0], r1(Db[0]), Ew[0], r1(Eb[0]))
    ef, ce2 = _bond_edge_call(e, bcat, Cw[0], r1(Cb[0]))

    for l in range(NLAYER):
        last = (l == NLAYER - 1)
        bdtab = bd.reshape(2 * NN, 128)
        if last:
            (nd,) = _sc_edge_pass(bdtab, eh, ce2, icat, True)
        else:
            e2, nd, est = _sc_edge_pass(bdtab, eh, ce2, icat, False)
        hn, ps = _hnew_call(nd, ah)
        if last:
            (y,) = _final_call(hn, hf, ps, r1(bnh_g[l]), r1(bnh_b[l]),
                               W0, r1(b0), W1, r1(b1), W2, r1(b2))
        else:
            hf, ah, bd, eh = _nodeup_call(
                hn, hf, ps, r1(bnh_g[l]), r1(bnh_b[l]),
                Aw[l + 1], r1(Ab[l + 1]), Bw[l + 1], r1(Bb[l + 1]),
                Dw[l + 1], r1(Db[l + 1]), Ew[l + 1], r1(Eb[l + 1]))
            est3 = est.reshape(2, NS, 128)
            if l < NLAYER - 2:
                ef, ce2 = _edgeup_call(e2, ef, est3, r1(bne_g[l]),
                                       r1(bne_b[l]), Cw[l + 1], r1(Cb[l + 1]),
                                       True)
            else:
                (ce2,) = _edgeup_call(e2, ef, est3, r1(bne_g[l]),
                                      r1(bne_b[l]), Cw[l + 1], r1(Cb[l + 1]),
                                      False)
    return y


# speedup vs baseline: 2.1648x; 1.0006x over previous
import functools
import jax
import jax.numpy as jnp
import numpy as np
from jax import lax
from jax.experimental import pallas as pl
from jax.experimental.pallas import tpu as pltpu
from jax.experimental.pallas import tpu_sc as plsc

NN = 10000
NEDGE = 320000
DIM = 128
NLAYER = 4
NC = 2
NS = 16
EC = 40
EPT = NEDGE // NS
NCHUNK = EPT // EC
NHALF = NCHUNK // 2
ROWS_PT = NN // NS
BN_EPS = 1e-5
DEN_EPS = 1e-6

BN_NODE = 1000
BN_GRID = NN // BN_NODE
BE_EDGE = 4000
BE_GRID = NEDGE // BE_EDGE


def _sc_edge_body(last, icat_h, bdtab_h, ehtab_h, ce_h,
                  *refs):
    if last:
        (nd_h,
         idxv0, idxv1, bdv0, bdv1, ehv0, ehv1, cev0, cev1,
         env0, env1, statv, zbuf, acc,
         sbd0, sbd1, seh0, seh1, sce0, sce1, sen0, sen1) = refs
        e2_h = None
        stats_h = None
    else:
        (e2_h, nd_h, stats_h,
         idxv0, idxv1, bdv0, bdv1, ehv0, ehv1, cev0, cev1,
         env0, env1, statv, zbuf, acc,
         sbd0, sbd1, seh0, seh1, sce0, sce1, sen0, sen1) = refs
    c = lax.axis_index("c")
    s = lax.axis_index("s")

    zero = jnp.zeros((16,), jnp.float32)

    def _zrow(i, carry):
        for q in range(8):
            zbuf[i, pl.ds(q * 16, 16)] = zero
        return carry
    lax.fori_loop(0, 40, _zrow, 0)

    @pl.when(s < 10)
    def _zero_acc():
        for kk in range(25):
            pltpu.sync_copy(zbuf, acc.at[pl.ds(s * 1000 + kk * 40, 40)])
    plsc.subcore_barrier()

    def _load_idx(k, idxv):
        pltpu.sync_copy(icat_h.at[c * (NS * NCHUNK) + s * NCHUNK + k], idxv)

    def _issue(k, idxv, bdv, ehv, cev, sbd, seh, sce):
        base = s * EPT + k * EC
        pltpu.async_copy(bdtab_h.at[idxv.at[0]], bdv, sbd)
        pltpu.async_copy(ehtab_h.at[idxv.at[1]], ehv, seh)
        pltpu.async_copy(ce_h.at[c, pl.ds(base, EC)], cev, sce)

    def _drain(bdv, ehv, cev, sbd, seh, sce):
        pltpu.make_async_copy(bdtab_h.at[idxv0.at[0]], bdv, sbd).wait()
        pltpu.make_async_copy(ehtab_h.at[idxv0.at[1]], ehv, seh).wait()
        pltpu.make_async_copy(ce_h.at[c, pl.ds(0, EC)], cev, sce).wait()

    def _compute(k, idxv, bdv, ehv, cev, env, cr):
        def _erow(j, cr2):
            cr2 = list(cr2)
            for q in range(4):
                dq = bdv[j, pl.ds(64 + q * 16, 16)]
                ehq = ehv[j, pl.ds(c * 64 + q * 16, 16)]
                ceq = cev[j, pl.ds(q * 16, 16)]
                ev = ceq + dq + ehq
                if not last:
                    env[j, pl.ds(q * 16, 16)] = ev
                    cr2[q] = cr2[q] + ev
                    cr2[4 + q] = cr2[4 + q] + ev * ev
                sig = 1.0 / (1.0 + jnp.exp(-ev))
                bq = bdv[j, pl.ds(q * 16, 16)]
                bdv[j, pl.ds(q * 16, 16)] = sig * bq
                bdv[j, pl.ds(64 + q * 16, 16)] = sig
            return tuple(cr2)

        cr = plsc.parallel_loop(0, EC, carry=cr, unroll=8)(_erow)
        pltpu.sync_copy(bdv, acc.at[idxv.at[1]], add=True)
        return cr

    _load_idx(0, idxv0)
    _issue(0, idxv0, bdv0, ehv0, cev0, sbd0, seh0, sce0)

    def _env_issue(k, env, sen):
        base = s * EPT + k * EC
        pltpu.async_copy(env, e2_h.at[c, pl.ds(base, EC)], sen)

    def _env_drain(env, sen):
        pltpu.make_async_copy(env, e2_h.at[c, pl.ds(0, EC)], sen).wait()

    def _pair(kp, cr):
        ka = 2 * kp
        _load_idx(ka + 1, idxv1)
        _issue(ka + 1, idxv1, bdv1, ehv1, cev1, sbd1, seh1, sce1)
        _drain(bdv0, ehv0, cev0, sbd0, seh0, sce0)
        if not last:
            @pl.when(kp > 0)
            def _():
                _env_drain(env0, sen0)
        cr = _compute(ka, idxv0, bdv0, ehv0, cev0, env0, cr)
        if not last:
            _env_issue(ka, env0, sen0)

        @pl.when(kp + 1 < NHALF)
        def _prefetch_next():
            _load_idx(ka + 2, idxv0)
            _issue(ka + 2, idxv0, bdv0, ehv0, cev0, sbd0, seh0, sce0)
        _drain(bdv1, ehv1, cev1, sbd1, seh1, sce1)
        if not last:
            @pl.when(kp > 0)
            def _():
                _env_drain(env1, sen1)
        cr = _compute(ka + 1, idxv1, bdv1, ehv1, cev1, env1, cr)
        if not last:
            _env_issue(ka + 1, env1, sen1)
        return cr

    init = tuple(zero for _ in range(8))
    st = lax.fori_loop(0, NHALF, _pair, init)
    if not last:
        _env_drain(env0, sen0)
        _env_drain(env1, sen1)

    if not last:
        for q in range(4):
            statv[pl.ds(q * 16, 16)] = st[q]
            statv[pl.ds(64 + q * 16, 16)] = st[4 + q]
        pltpu.sync_copy(statv, stats_h.at[pl.ds((c * NS + s) * 128, 128)])

    plsc.subcore_barrier()

    @pl.when(s < 10)
    def _dump_acc():
        pltpu.sync_copy(acc.at[pl.ds(s * 1000, 1000)],
                        nd_h.at[c, pl.ds(s * 1000, 1000)])


def _sc_edge_pass(bdtab, ehtab, ce2, icat, last):
    mesh = plsc.VectorSubcoreMesh(core_axis_name="c", subcore_axis_name="s",
                                  num_cores=NC, num_subcores=NS)
    if last:
        out_type = (jax.ShapeDtypeStruct((NC, NN, 128), jnp.float32),)
    else:
        out_type = (
            jax.ShapeDtypeStruct((NC, NEDGE, 64), jnp.float32),
            jax.ShapeDtypeStruct((NC, NN, 128), jnp.float32),
            jax.ShapeDtypeStruct((NC * NS * 128,), jnp.float32),
        )
    scratch = [
        pltpu.VMEM((2, EC), jnp.int32),
        pltpu.VMEM((2, EC), jnp.int32),
        pltpu.VMEM((EC, 128), jnp.float32),
        pltpu.VMEM((EC, 128), jnp.float32),
        pltpu.VMEM((EC, 128), jnp.float32),
        pltpu.VMEM((EC, 128), jnp.float32),
        pltpu.VMEM((EC, 64), jnp.float32),
        pltpu.VMEM((EC, 64), jnp.float32),
        pltpu.VMEM((EC, 64), jnp.float32),
        pltpu.VMEM((EC, 64), jnp.float32),
        pltpu.VMEM((128,), jnp.float32),
        pltpu.VMEM((40, 128), jnp.float32),
        pltpu.VMEM_SHARED((NN, 128), jnp.float32),
    ] + [pltpu.SemaphoreType.DMA] * 8
    fn = pl.kernel(functools.partial(_sc_edge_body, last),
                   out_type=out_type, mesh=mesh, scratch_types=scratch)
    return fn(icat, bdtab, ehtab, ce2)


def _node_mats(hf, wa, ab, wb, bb, wd, db, we, eb,
               hf_out, ah_out, bd_out, eh_out):
    if hf_out is not None:
        hf_out[...] = hf
    f32 = jnp.float32
    Ah = jnp.dot(hf, wa[...], preferred_element_type=f32) + ab[...]
    Bh = jnp.dot(hf, wb[...], preferred_element_type=f32) + bb[...]
    Dh = jnp.dot(hf, wd[...], preferred_element_type=f32) + db[...]
    Eh = jnp.dot(hf, we[...], preferred_element_type=f32) + eb[...]
    ah_out[...] = Ah
    bd_out[0] = jnp.concatenate([Bh[:, :64], Dh[:, :64]], axis=1)
    bd_out[1] = jnp.concatenate([Bh[:, 64:], Dh[:, 64:]], axis=1)
    eh_out[...] = Eh


def _enc_node_body(h_ref, p_ref, acat, pw, pb2,
                   wa, ab, wb, bb, wd, db, we, eb,
                   hf_out, ah_out, bd_out, eh_out):
    f32 = jnp.float32
    hf = jnp.dot(p_ref[...], pw[...], preferred_element_type=f32) + pb2[...]
    iot = lax.broadcasted_iota(jnp.int32, (1, 128), 1)
    for i in range(9):
        oh = (h_ref[:, i:i + 1] == iot).astype(f32)
        hf = hf + jnp.dot(oh, acat[i], preferred_element_type=f32)
    _node_mats(hf, wa, ab, wb, bb, wd, db, we, eb,
               hf_out, ah_out, bd_out, eh_out)


def _enc_node_call(h, p, acat, pw, pb2, wa, ab, wb, bb, wd, db, we, eb):
    B = BN_NODE
    full = lambda *shape: pl.BlockSpec(shape, lambda i: (0,) * len(shape))
    blk = lambda *shape: pl.BlockSpec(shape, lambda i: (i,) + (0,) * (len(shape) - 1))
    blk2 = lambda *shape: pl.BlockSpec(shape, lambda i: (0, i) + (0,) * (len(shape) - 2))
    return pl.pallas_call(
        _enc_node_body,
        grid=(BN_GRID,),
        in_specs=[blk(B, 9), blk(B, 8), full(9, 128, 128), full(8, 128),
                  full(1, 128)] + [full(128, 128), full(1, 128)] * 4,
        out_specs=[blk(B, 128), blk(B, 128), blk2(2, B, 128), blk(B, 128)],
        out_shape=[
            jax.ShapeDtypeStruct((NN, 128), jnp.float32),
            jax.ShapeDtypeStruct((NN, 128), jnp.float32),
            jax.ShapeDtypeStruct((2, NN, 128), jnp.float32),
            jax.ShapeDtypeStruct((NN, 128), jnp.float32),
        ],
    )(h, p, acat, pw, pb2, wa, ab, wb, bb, wd, db, we, eb)


def _bond_edge_body(e_ref, bcat, cw, cb2, ef_out, ce_out):
    f32 = jnp.float32
    iot = lax.broadcasted_iota(jnp.int32, (1, 8), 1)
    ef = jnp.zeros((e_ref.shape[0], 128), f32)
    for i in range(3):
        oh = (e_ref[:, i:i + 1] == iot).astype(f32)
        ef = ef + jnp.dot(oh, bcat[i], preferred_element_type=f32)
    ef_out[...] = ef
    ce = jnp.dot(ef, cw[...], preferred_element_type=f32) + cb2[...]
    ce_out[0] = ce[:, :64]
    ce_out[1] = ce[:, 64:]


def _bond_edge_call(e, bcat, cw, cb2):
    B = BE_EDGE
    full = lambda *shape: pl.BlockSpec(shape, lambda i: (0,) * len(shape))
    blk = lambda *shape: pl.BlockSpec(shape, lambda i: (i,) + (0,) * (len(shape) - 1))
    blk2 = lambda *shape: pl.BlockSpec(shape, lambda i: (0, i) + (0,) * (len(shape) - 2))
    return pl.pallas_call(
        _bond_edge_body,
        grid=(BE_GRID,),
        in_specs=[blk(B, 3), full(3, 8, 128), full(128, 128), full(1, 128)],
        out_specs=[blk(B, 128), blk2(2, B, 64)],
        out_shape=[
            jax.ShapeDtypeStruct((NEDGE, 128), jnp.float32),
            jax.ShapeDtypeStruct((2, NEDGE, 64), jnp.float32),
        ],
    )(e, bcat, cw, cb2)


def _hnew_body(nd_ref, ah_ref, hn_out, ps_out):
    num = jnp.concatenate([nd_ref[0][:, :64], nd_ref[1][:, :64]], axis=1)
    den = jnp.concatenate([nd_ref[0][:, 64:], nd_ref[1][:, 64:]], axis=1)
    hn = ah_ref[...] + num / (den + DEN_EPS)
    hn_out[...] = hn
    s1 = jnp.sum(hn, axis=0, keepdims=True)
    s2 = jnp.sum(hn * hn, axis=0, keepdims=True)
    ps_out[...] = jnp.concatenate(
        [s1, s2, jnp.zeros((6, 128), jnp.float32)], axis=0).reshape(1, 8, 128)


def _hnew_call(nd, ah):
    B = BN_NODE
    blk = lambda *shape: pl.BlockSpec(shape, lambda i: (i,) + (0,) * (len(shape) - 1))
    blk2 = lambda *shape: pl.BlockSpec(shape, lambda i: (0, i) + (0,) * (len(shape) - 2))
    return pl.pallas_call(
        _hnew_body,
        grid=(BN_GRID,),
        in_specs=[blk2(2, B, 128), blk(B, 128)],
        out_specs=[blk(B, 128), blk(1, 8, 128)],
        out_shape=[
            jax.ShapeDtypeStruct((NN, 128), jnp.float32),
            jax.ShapeDtypeStruct((BN_GRID, 8, 128), jnp.float32),
        ],
    )(nd, ah)


def _node_stats(ps_ref):
    m = jnp.sum(ps_ref[:, 0, :], axis=0, keepdims=True) / NN
    v = jnp.sum(ps_ref[:, 1, :], axis=0, keepdims=True) / NN - m * m
    return m, v


def _nodeup_body(hn_ref, hfp_ref, ps_ref, g2, b2,
                 wa, ab, wb, bb, wd, db, we, eb,
                 hf_out, ah_out, bd_out, eh_out):
    m, v = _node_stats(ps_ref)
    bn = g2[...] * (hn_ref[...] - m) * lax.rsqrt(v + BN_EPS) + b2[...]
    hf = hfp_ref[...] + jnp.maximum(bn, 0.0)
    _node_mats(hf, wa, ab, wb, bb, wd, db, we, eb,
               hf_out, ah_out, bd_out, eh_out)


def _nodeup_call(hn, hfp, ps, g2, b2, wa, ab, wb, bb, wd, db, we, eb):
    B = BN_NODE
    full = lambda *shape: pl.BlockSpec(shape, lambda i: (0,) * len(shape))
    blk = lambda *shape: pl.BlockSpec(shape, lambda i: (i,) + (0,) * (len(shape) - 1))
    blk2 = lambda *shape: pl.BlockSpec(shape, lambda i: (0, i) + (0,) * (len(shape) - 2))
    return pl.pallas_call(
        _nodeup_body,
        grid=(BN_GRID,),
        in_specs=[blk(B, 128), blk(B, 128), full(BN_GRID, 8, 128),
                  full(1, 128), full(1, 128)]
                 + [full(128, 128), full(1, 128)] * 4,
        out_specs=[blk(B, 128), blk(B, 128), blk2(2, B, 128), blk(B, 128)],
        out_shape=[
            jax.ShapeDtypeStruct((NN, 128), jnp.float32),
            jax.ShapeDtypeStruct((NN, 128), jnp.float32),
            jax.ShapeDtypeStruct((2, NN, 128), jnp.float32),
            jax.ShapeDtypeStruct((NN, 128), jnp.float32),
        ],
    )(hn, hfp, ps, g2, b2, wa, ab, wb, bb, wd, db, we, eb)


def _edge_stats(est_ref):
    e0 = jnp.sum(est_ref[0], axis=0)
    e1 = jnp.sum(est_ref[1], axis=0)
    m = jnp.concatenate([e0[:64], e1[:64]]).reshape(1, 128) / NEDGE
    q = jnp.concatenate([e0[64:], e1[64:]]).reshape(1, 128) / NEDGE
    return m, q - m * m


def _edgeup_body(write_ef, e2_ref, ef_ref, est_ref, g2, b2, cw, cb2, *outs):
    if write_ef:
        ef_out, ce_out = outs
    else:
        (ce_out,) = outs
    m, v = _edge_stats(est_ref)
    en = jnp.concatenate([e2_ref[0], e2_ref[1]], axis=1)
    bn = g2[...] * (en - m) * lax.rsqrt(v + BN_EPS) + b2[...]
    ef = ef_ref[...] + jnp.maximum(bn, 0.0)
    if write_ef:
        ef_out[...] = ef
    ce = jnp.dot(ef, cw[...], preferred_element_type=jnp.float32) + cb2[...]
    ce_out[0] = ce[:, :64]
    ce_out[1] = ce[:, 64:]


def _edgeup_call(e2, ef, est, g2, b2, cw, cb2, write_ef):
    B = BE_EDGE
    full = lambda *shape: pl.BlockSpec(shape, lambda i: (0,) * len(shape))
    blk = lambda *shape: pl.BlockSpec(shape, lambda i: (i,) + (0,) * (len(shape) - 1))
    blk2 = lambda *shape: pl.BlockSpec(shape, lambda i: (0, i) + (0,) * (len(shape) - 2))
    out_specs = [blk2(2, B, 64)]
    out_shape = [jax.ShapeDtypeStruct((2, NEDGE, 64), jnp.float32)]
    if write_ef:
        out_specs = [blk(B, 128)] + out_specs
        out_shape = [jax.ShapeDtypeStruct((NEDGE, 128), jnp.float32)] + out_shape
    return pl.pallas_call(
        functools.partial(_edgeup_body, write_ef),
        grid=(BE_GRID,),
        in_specs=[blk2(2, B, 64), blk(B, 128), full(2, NS, 128),
                  full(1, 128), full(1, 128), full(128, 128), full(1, 128)],
        out_specs=out_specs,
        out_shape=out_shape,
    )(e2, ef, est, g2, b2, cw, cb2)


def _final_body(hn_ref, hfp_ref, ps_ref, g2, b2,
                w0, b0r, w1, b1r, w2, b2r, y_out):
    m, v = _node_stats(ps_ref)
    bn = g2[...] * (hn_ref[...] - m) * lax.rsqrt(v + BN_EPS) + b2[...]
    hf = hfp_ref[...] + jnp.maximum(bn, 0.0)
    hg = jnp.sum(hf, axis=0, keepdims=True) / NN
    f32 = jnp.float32
    y = jnp.maximum(jnp.dot(hg, w0[...], preferred_element_type=f32) + b0r[...], 0.0)
    y = jnp.maximum(jnp.dot(y, w1[...], preferred_element_type=f32) + b1r[...], 0.0)
    y = jnp.dot(y, w2[...], preferred_element_type=f32) + b2r[...]
    y_out[...] = y


def _final_call(hn, hfp, ps, g2, b2, w0, b0r, w1, b1r, w2, b2r):
    full = lambda *shape: pl.BlockSpec(shape, lambda i: (0,) * len(shape))
    return pl.pallas_call(
        _final_body,
        grid=(1,),
        in_specs=[full(NN, 128), full(NN, 128), full(BN_GRID, 8, 128),
                  full(1, 128), full(1, 128), full(128, 64), full(1, 64),
                  full(64, 32), full(1, 32), full(32, 128), full(1, 128)],
        out_specs=[full(1, 128)],
        out_shape=[jax.ShapeDtypeStruct((1, 128), jnp.float32)],
    )(hn, hfp, ps, g2, b2, w0, b0r, w1, b1r, w2, b2r)


def kernel(h, p, e, snorm_n, edge_index,
           atom_emb_0, atom_emb_1, atom_emb_2, atom_emb_3, atom_emb_4,
           atom_emb_5, atom_emb_6, atom_emb_7, atom_emb_8,
           bond_emb_0, bond_emb_1, bond_emb_2,
           Pw, Pb, Aw, Ab, Bw, Bb, Cw, Cb, Dw, Db, Ew, Eb,
           bnh_g, bnh_b, bne_g, bne_b, W0, b0, W1, b1, W2, b2):
    f32 = jnp.float32
    atom_tabs = [atom_emb_0, atom_emb_1, atom_emb_2, atom_emb_3, atom_emb_4,
                 atom_emb_5, atom_emb_6, atom_emb_7, atom_emb_8]
    acat = jnp.stack([jnp.pad(t, ((0, 128 - t.shape[0]), (0, 0)))
                      for t in atom_tabs])
    bond_tabs = [bond_emb_0, bond_emb_1, bond_emb_2]
    bcat = jnp.stack([jnp.pad(t, ((0, 8 - t.shape[0]), (0, 0)))
                      for t in bond_tabs])
    r1 = lambda x: x.reshape(1, -1).astype(f32)

    src = edge_index[0].astype(jnp.int32)
    dst = edge_index[1].astype(jnp.int32)
    src2 = jnp.concatenate([src, src + NN])
    src2r = src2.reshape(2, NS, NCHUNK, EC)
    dstr = jnp.broadcast_to(dst.reshape(1, NS, NCHUNK, EC),
                            (2, NS, NCHUNK, EC))
    icat = jnp.stack([src2r, dstr], axis=3).reshape(2 * NS * NCHUNK, 2, EC)

    h = h.astype(jnp.int32)
    e = e.astype(jnp.int32)

    hf, ah, bd, eh = _enc_node_call(
        h, p.astype(f32), acat, Pw, r1(Pb),
        Aw[0], r1(Ab[0]), Bw[0], r1(Bb[0]), Dw[0], r1(Db[0]), Ew[0], r1(Eb[0]))
    ef, ce2 = _bond_edge_call(e, bcat, Cw[0], r1(Cb[0]))

    for l in range(NLAYER):
        last = (l == NLAYER - 1)
        bdtab = bd.reshape(2 * NN, 128)
        if last:
            (nd,) = _sc_edge_pass(bdtab, eh, ce2, icat, True)
        else:
            e2, nd, est = _sc_edge_pass(bdtab, eh, ce2, icat, False)
        hn, ps = _hnew_call(nd, ah)
        if last:
            (y,) = _final_call(hn, hf, ps, r1(bnh_g[l]), r1(bnh_b[l]),
                               W0, r1(b0), W1, r1(b1), W2, r1(b2))
        else:
            hf, ah, bd, eh = _nodeup_call(
                hn, hf, ps, r1(bnh_g[l]), r1(bnh_b[l]),
                Aw[l + 1], r1(Ab[l + 1]), Bw[l + 1], r1(Bb[l + 1]),
                Dw[l + 1], r1(Db[l + 1]), Ew[l + 1], r1(Eb[l + 1]))
            est3 = est.reshape(2, NS, 128)
            if l < NLAYER - 2:
                ef, ce2 = _edgeup_call(e2, ef, est3, r1(bne_g[l]),
                                       r1(bne_b[l]), Cw[l + 1], r1(Cb[l + 1]),
                                       True)
            else:
                (ce2,) = _edgeup_call(e2, ef, est3, r1(bne_g[l]),
                                      r1(bne_b[l]), Cw[l + 1], r1(Cb[l + 1]),
                                      False)
    return y


# speedup vs baseline: 4.0509x; 1.8713x over previous
import functools
import jax
import jax.numpy as jnp
import numpy as np
from jax import lax
from jax.experimental import pallas as pl
from jax.experimental.pallas import tpu as pltpu
from jax.experimental.pallas import tpu_sc as plsc

NN = 10000
NEDGE = 320000
DIM = 128
NLAYER = 4
NC = 2
NS = 16
EC = 40
EPT = NEDGE // NS
NCHUNK = EPT // EC
NHALF = NCHUNK // 2
ROWS_PT = NN // NS
BN_EPS = 1e-5
DEN_EPS = 1e-6

BN_NODE = 1000
BN_GRID = NN // BN_NODE
BE_EDGE = 8000
BE_GRID = NEDGE // BE_EDGE


def _sc_edge_body(last, icat_h, bdtab_h, ehtab_h, ce_h,
                  *refs):
    if last:
        (nd_h,
         idxv0, idxv1, bdv0, bdv1, ehv0, ehv1, cev0, cev1,
         env0, env1, zbuf, acc,
         sbd0, sbd1, seh0, seh1, sce0, sce1, sen0, sen1) = refs
        e2_h = None
    else:
        (e2_h, nd_h,
         idxv0, idxv1, bdv0, bdv1, ehv0, ehv1, cev0, cev1,
         env0, env1, zbuf, acc,
         sbd0, sbd1, seh0, seh1, sce0, sce1, sen0, sen1) = refs
    c = lax.axis_index("c")
    s = lax.axis_index("s")

    zero = jnp.zeros((16,), jnp.float32)

    def _zrow(i, carry):
        for q in range(8):
            zbuf[i, pl.ds(q * 16, 16)] = zero
        return carry
    lax.fori_loop(0, 40, _zrow, 0)

    @pl.when(s < 10)
    def _zero_acc():
        for kk in range(25):
            pltpu.sync_copy(zbuf, acc.at[pl.ds(s * 1000 + kk * 40, 40)])
    plsc.subcore_barrier()

    def _load_idx(k, idxv):
        pltpu.sync_copy(icat_h.at[c * (NS * NCHUNK) + s * NCHUNK + k], idxv)

    def _issue(k, idxv, bdv, ehv, cev, sbd, seh, sce):
        base = s * EPT + k * EC
        pltpu.async_copy(bdtab_h.at[idxv.at[0]], bdv, sbd)
        pltpu.async_copy(ehtab_h.at[idxv.at[1]], ehv, seh)
        pltpu.async_copy(ce_h.at[c, pl.ds(base, EC)], cev, sce)

    def _drain(bdv, ehv, cev, sbd, seh, sce):
        pltpu.make_async_copy(bdtab_h.at[idxv0.at[0]], bdv, sbd).wait()
        pltpu.make_async_copy(ehtab_h.at[idxv0.at[1]], ehv, seh).wait()
        pltpu.make_async_copy(ce_h.at[c, pl.ds(0, EC)], cev, sce).wait()

    def _compute(k, idxv, bdv, ehv, cev, env):
        def _erow(j):
            for q in range(4):
                dq = bdv[j, pl.ds(64 + q * 16, 16)]
                ehq = ehv[j, pl.ds(c * 64 + q * 16, 16)]
                ceq = cev[j, pl.ds(q * 16, 16)]
                ev = ceq + dq + ehq
                if not last:
                    env[j, pl.ds(q * 16, 16)] = ev
                sig = 1.0 / (1.0 + jnp.exp(-ev))
                bq = bdv[j, pl.ds(q * 16, 16)]
                bdv[j, pl.ds(q * 16, 16)] = sig * bq
                bdv[j, pl.ds(64 + q * 16, 16)] = sig

        plsc.parallel_loop(0, EC, unroll=(8 if last else 4))(_erow)
        pltpu.sync_copy(bdv, acc.at[idxv.at[1]], add=True)

    _load_idx(0, idxv0)
    _issue(0, idxv0, bdv0, ehv0, cev0, sbd0, seh0, sce0)

    def _env_issue(k, env, sen):
        base = s * EPT + k * EC
        pltpu.async_copy(env, e2_h.at[c, pl.ds(base, EC)], sen)

    def _env_drain(env, sen):
        pltpu.make_async_copy(env, e2_h.at[c, pl.ds(0, EC)], sen).wait()

    def _pair(kp, cr):
        ka = 2 * kp
        _load_idx(ka + 1, idxv1)
        _issue(ka + 1, idxv1, bdv1, ehv1, cev1, sbd1, seh1, sce1)
        _drain(bdv0, ehv0, cev0, sbd0, seh0, sce0)
        if not last:
            @pl.when(kp > 0)
            def _():
                _env_drain(env0, sen0)
        _compute(ka, idxv0, bdv0, ehv0, cev0, env0)
        if not last:
            _env_issue(ka, env0, sen0)

        @pl.when(kp + 1 < NHALF)
        def _prefetch_next():
            _load_idx(ka + 2, idxv0)
            _issue(ka + 2, idxv0, bdv0, ehv0, cev0, sbd0, seh0, sce0)
        _drain(bdv1, ehv1, cev1, sbd1, seh1, sce1)
        if not last:
            @pl.when(kp > 0)
            def _():
                _env_drain(env1, sen1)
        _compute(ka + 1, idxv1, bdv1, ehv1, cev1, env1)
        if not last:
            _env_issue(ka + 1, env1, sen1)
        return cr

    lax.fori_loop(0, NHALF, _pair, 0)
    if not last:
        _env_drain(env0, sen0)
        _env_drain(env1, sen1)

    plsc.subcore_barrier()

    @pl.when(s < 10)
    def _dump_acc():
        pltpu.sync_copy(acc.at[pl.ds(s * 1000, 1000)],
                        nd_h.at[c, pl.ds(s * 1000, 1000)])


def _sc_edge_pass(bdtab, ehtab, ce2, icat, last):
    mesh = plsc.VectorSubcoreMesh(core_axis_name="c", subcore_axis_name="s",
                                  num_cores=NC, num_subcores=NS)
    if last:
        out_type = (jax.ShapeDtypeStruct((NC, NN, 128), jnp.float32),)
    else:
        out_type = (
            jax.ShapeDtypeStruct((NC, NEDGE, 64), jnp.float32),
            jax.ShapeDtypeStruct((NC, NN, 128), jnp.float32),
        )
    scratch = [
        pltpu.VMEM((2, EC), jnp.int32),
        pltpu.VMEM((2, EC), jnp.int32),
        pltpu.VMEM((EC, 128), jnp.float32),
        pltpu.VMEM((EC, 128), jnp.float32),
        pltpu.VMEM((EC, 128), jnp.float32),
        pltpu.VMEM((EC, 128), jnp.float32),
        pltpu.VMEM((EC, 64), jnp.float32),
        pltpu.VMEM((EC, 64), jnp.float32),
        pltpu.VMEM((EC, 64), jnp.float32),
        pltpu.VMEM((EC, 64), jnp.float32),
        pltpu.VMEM((40, 128), jnp.float32),
        pltpu.VMEM_SHARED((NN, 128), jnp.float32),
    ] + [pltpu.SemaphoreType.DMA] * 8
    fn = pl.kernel(functools.partial(_sc_edge_body, last),
                   out_type=out_type, mesh=mesh, scratch_types=scratch)
    return fn(icat, bdtab, ehtab, ce2)


def _node_mats(hf, wa, ab, wb, bb, wd, db, we, eb,
               hf_out, ah_out, bd_out, eh_out):
    if hf_out is not None:
        hf_out[...] = hf
    f32 = jnp.float32
    Ah = jnp.dot(hf, wa[...], preferred_element_type=f32) + ab[...]
    Bh = jnp.dot(hf, wb[...], preferred_element_type=f32) + bb[...]
    Dh = jnp.dot(hf, wd[...], preferred_element_type=f32) + db[...]
    Eh = jnp.dot(hf, we[...], preferred_element_type=f32) + eb[...]
    ah_out[...] = Ah
    bd_out[0] = jnp.concatenate([Bh[:, :64], Dh[:, :64]], axis=1)
    bd_out[1] = jnp.concatenate([Bh[:, 64:], Dh[:, 64:]], axis=1)
    eh_out[...] = Eh


def _enc_node_body(h_ref, p_ref, acat, pw, pb2,
                   wa, ab, wb, bb, wd, db, we, eb,
                   hf_out, ah_out, bd_out, eh_out):
    f32 = jnp.float32
    hf = jnp.dot(p_ref[...], pw[...], preferred_element_type=f32) + pb2[...]
    iot = lax.broadcasted_iota(jnp.int32, (1, 128), 1)
    for i in range(9):
        oh = (h_ref[:, i:i + 1] == iot).astype(f32)
        hf = hf + jnp.dot(oh, acat[i], preferred_element_type=f32)
    _node_mats(hf, wa, ab, wb, bb, wd, db, we, eb,
               hf_out, ah_out, bd_out, eh_out)


def _enc_node_call(h, p, acat, pw, pb2, wa, ab, wb, bb, wd, db, we, eb):
    B = BN_NODE
    full = lambda *shape: pl.BlockSpec(shape, lambda i: (0,) * len(shape))
    blk = lambda *shape: pl.BlockSpec(shape, lambda i: (i,) + (0,) * (len(shape) - 1))
    blk2 = lambda *shape: pl.BlockSpec(shape, lambda i: (0, i) + (0,) * (len(shape) - 2))
    return pl.pallas_call(
        _enc_node_body,
        grid=(BN_GRID,),
        in_specs=[blk(B, 9), blk(B, 8), full(9, 128, 128), full(8, 128),
                  full(1, 128)] + [full(128, 128), full(1, 128)] * 4,
        out_specs=[blk(B, 128), blk(B, 128), blk2(2, B, 128), blk(B, 128)],
        out_shape=[
            jax.ShapeDtypeStruct((NN, 128), jnp.float32),
            jax.ShapeDtypeStruct((NN, 128), jnp.float32),
            jax.ShapeDtypeStruct((2, NN, 128), jnp.float32),
            jax.ShapeDtypeStruct((NN, 128), jnp.float32),
        ],
    )(h, p, acat, pw, pb2, wa, ab, wb, bb, wd, db, we, eb)


def _bond_edge_body(e_ref, bcat, cw, cb2, ef_out, ce_out):
    f32 = jnp.float32
    iot = lax.broadcasted_iota(jnp.int32, (1, 8), 1)
    ef = jnp.zeros((e_ref.shape[0], 128), f32)
    for i in range(3):
        oh = (e_ref[:, i:i + 1] == iot).astype(f32)
        ef = ef + jnp.dot(oh, bcat[i], preferred_element_type=f32)
    ef_out[...] = ef
    ce = jnp.dot(ef, cw[...], preferred_element_type=f32) + cb2[...]
    ce_out[0] = ce[:, :64]
    ce_out[1] = ce[:, 64:]


def _bond_edge_call(e, bcat, cw, cb2):
    B = BE_EDGE
    full = lambda *shape: pl.BlockSpec(shape, lambda i: (0,) * len(shape))
    blk = lambda *shape: pl.BlockSpec(shape, lambda i: (i,) + (0,) * (len(shape) - 1))
    blk2 = lambda *shape: pl.BlockSpec(shape, lambda i: (0, i) + (0,) * (len(shape) - 2))
    return pl.pallas_call(
        _bond_edge_body,
        grid=(BE_GRID,),
        in_specs=[blk(B, 3), full(3, 8, 128), full(128, 128), full(1, 128)],
        out_specs=[blk(B, 128), blk2(2, B, 64)],
        out_shape=[
            jax.ShapeDtypeStruct((NEDGE, 128), jnp.float32),
            jax.ShapeDtypeStruct((2, NEDGE, 64), jnp.float32),
        ],
    )(e, bcat, cw, cb2)


def _hnew_body(nd_ref, ah_ref, hn_out, ps_out):
    num = jnp.concatenate([nd_ref[0][:, :64], nd_ref[1][:, :64]], axis=1)
    den = jnp.concatenate([nd_ref[0][:, 64:], nd_ref[1][:, 64:]], axis=1)
    hn = ah_ref[...] + num / (den + DEN_EPS)
    hn_out[...] = hn
    s1 = jnp.sum(hn, axis=0, keepdims=True)
    s2 = jnp.sum(hn * hn, axis=0, keepdims=True)
    ps_out[...] = jnp.concatenate(
        [s1, s2, jnp.zeros((6, 128), jnp.float32)], axis=0).reshape(1, 8, 128)


def _hnew_call(nd, ah):
    B = BN_NODE
    blk = lambda *shape: pl.BlockSpec(shape, lambda i: (i,) + (0,) * (len(shape) - 1))
    blk2 = lambda *shape: pl.BlockSpec(shape, lambda i: (0, i) + (0,) * (len(shape) - 2))
    return pl.pallas_call(
        _hnew_body,
        grid=(BN_GRID,),
        in_specs=[blk2(2, B, 128), blk(B, 128)],
        out_specs=[blk(B, 128), blk(1, 8, 128)],
        out_shape=[
            jax.ShapeDtypeStruct((NN, 128), jnp.float32),
            jax.ShapeDtypeStruct((BN_GRID, 8, 128), jnp.float32),
        ],
    )(nd, ah)


def _node_stats(ps_ref):
    m = jnp.sum(ps_ref[:, 0, :], axis=0, keepdims=True) / NN
    v = jnp.sum(ps_ref[:, 1, :], axis=0, keepdims=True) / NN - m * m
    return m, v


def _nodeup_body(hn_ref, hfp_ref, ps_ref, g2, b2,
                 wa, ab, wb, bb, wd, db, we, eb,
                 hf_out, ah_out, bd_out, eh_out):
    m, v = _node_stats(ps_ref)
    bn = g2[...] * (hn_ref[...] - m) * lax.rsqrt(v + BN_EPS) + b2[...]
    hf = hfp_ref[...] + jnp.maximum(bn, 0.0)
    _node_mats(hf, wa, ab, wb, bb, wd, db, we, eb,
               hf_out, ah_out, bd_out, eh_out)


def _nodeup_call(hn, hfp, ps, g2, b2, wa, ab, wb, bb, wd, db, we, eb):
    B = BN_NODE
    full = lambda *shape: pl.BlockSpec(shape, lambda i: (0,) * len(shape))
    blk = lambda *shape: pl.BlockSpec(shape, lambda i: (i,) + (0,) * (len(shape) - 1))
    blk2 = lambda *shape: pl.BlockSpec(shape, lambda i: (0, i) + (0,) * (len(shape) - 2))
    return pl.pallas_call(
        _nodeup_body,
        grid=(BN_GRID,),
        in_specs=[blk(B, 128), blk(B, 128), full(BN_GRID, 8, 128),
                  full(1, 128), full(1, 128)]
                 + [full(128, 128), full(1, 128)] * 4,
        out_specs=[blk(B, 128), blk(B, 128), blk2(2, B, 128), blk(B, 128)],
        out_shape=[
            jax.ShapeDtypeStruct((NN, 128), jnp.float32),
            jax.ShapeDtypeStruct((NN, 128), jnp.float32),
            jax.ShapeDtypeStruct((2, NN, 128), jnp.float32),
            jax.ShapeDtypeStruct((NN, 128), jnp.float32),
        ],
    )(hn, hfp, ps, g2, b2, wa, ab, wb, bb, wd, db, we, eb)


def _estats_body(e2_ref, ps_out):
    en = jnp.concatenate([e2_ref[0], e2_ref[1]], axis=1)
    s1 = jnp.sum(en, axis=0, keepdims=True)
    s2 = jnp.sum(en * en, axis=0, keepdims=True)
    ps_out[...] = jnp.concatenate(
        [s1, s2, jnp.zeros((6, 128), jnp.float32)], axis=0).reshape(1, 8, 128)


def _estats_call(e2):
    B = BE_EDGE
    blk = lambda *shape: pl.BlockSpec(shape, lambda i: (i,) + (0,) * (len(shape) - 1))
    blk2 = lambda *shape: pl.BlockSpec(shape, lambda i: (0, i) + (0,) * (len(shape) - 2))
    return pl.pallas_call(
        _estats_body,
        grid=(BE_GRID,),
        in_specs=[blk2(2, B, 64)],
        out_specs=[blk(1, 8, 128)],
        out_shape=[jax.ShapeDtypeStruct((BE_GRID, 8, 128), jnp.float32)],
    )(e2)


def _edge_stats(est_ref):
    m = jnp.sum(est_ref[:, 0, :], axis=0, keepdims=True) / NEDGE
    v = jnp.sum(est_ref[:, 1, :], axis=0, keepdims=True) / NEDGE - m * m
    return m, v


def _edgeup_body(write_ef, e2_ref, ef_ref, est_ref, g2, b2, cw, cb2, *outs):
    if write_ef:
        ef_out, ce_out = outs
    else:
        (ce_out,) = outs
    m, v = _edge_stats(est_ref)
    en = jnp.concatenate([e2_ref[0], e2_ref[1]], axis=1)
    bn = g2[...] * (en - m) * lax.rsqrt(v + BN_EPS) + b2[...]
    ef = ef_ref[...] + jnp.maximum(bn, 0.0)
    if write_ef:
        ef_out[...] = ef
    ce = jnp.dot(ef, cw[...], preferred_element_type=jnp.float32) + cb2[...]
    ce_out[0] = ce[:, :64]
    ce_out[1] = ce[:, 64:]


def _edgeup_call(e2, ef, est, g2, b2, cw, cb2, write_ef):
    B = BE_EDGE
    full = lambda *shape: pl.BlockSpec(shape, lambda i: (0,) * len(shape))
    blk = lambda *shape: pl.BlockSpec(shape, lambda i: (i,) + (0,) * (len(shape) - 1))
    blk2 = lambda *shape: pl.BlockSpec(shape, lambda i: (0, i) + (0,) * (len(shape) - 2))
    out_specs = [blk2(2, B, 64)]
    out_shape = [jax.ShapeDtypeStruct((2, NEDGE, 64), jnp.float32)]
    if write_ef:
        out_specs = [blk(B, 128)] + out_specs
        out_shape = [jax.ShapeDtypeStruct((NEDGE, 128), jnp.float32)] + out_shape
    return pl.pallas_call(
        functools.partial(_edgeup_body, write_ef),
        grid=(BE_GRID,),
        in_specs=[blk2(2, B, 64), blk(B, 128), full(BE_GRID, 8, 128),
                  full(1, 128), full(1, 128), full(128, 128), full(1, 128)],
        out_specs=out_specs,
        out_shape=out_shape,
    )(e2, ef, est, g2, b2, cw, cb2)


def _final_body(hn_ref, hfp_ref, ps_ref, g2, b2,
                w0, b0r, w1, b1r, w2, b2r, y_out):
    m, v = _node_stats(ps_ref)
    bn = g2[...] * (hn_ref[...] - m) * lax.rsqrt(v + BN_EPS) + b2[...]
    hf = hfp_ref[...] + jnp.maximum(bn, 0.0)
    hg = jnp.sum(hf, axis=0, keepdims=True) / NN
    f32 = jnp.float32
    y = jnp.maximum(jnp.dot(hg, w0[...], preferred_element_type=f32) + b0r[...], 0.0)
    y = jnp.maximum(jnp.dot(y, w1[...], preferred_element_type=f32) + b1r[...], 0.0)
    y = jnp.dot(y, w2[...], preferred_element_type=f32) + b2r[...]
    y_out[...] = y


def _final_call(hn, hfp, ps, g2, b2, w0, b0r, w1, b1r, w2, b2r):
    full = lambda *shape: pl.BlockSpec(shape, lambda i: (0,) * len(shape))
    return pl.pallas_call(
        _final_body,
        grid=(1,),
        in_specs=[full(NN, 128), full(NN, 128), full(BN_GRID, 8, 128),
                  full(1, 128), full(1, 128), full(128, 64), full(1, 64),
                  full(64, 32), full(1, 32), full(32, 128), full(1, 128)],
        out_specs=[full(1, 128)],
        out_shape=[jax.ShapeDtypeStruct((1, 128), jnp.float32)],
    )(hn, hfp, ps, g2, b2, w0, b0r, w1, b1r, w2, b2r)


def kernel(h, p, e, snorm_n, edge_index,
           atom_emb_0, atom_emb_1, atom_emb_2, atom_emb_3, atom_emb_4,
           atom_emb_5, atom_emb_6, atom_emb_7, atom_emb_8,
           bond_emb_0, bond_emb_1, bond_emb_2,
           Pw, Pb, Aw, Ab, Bw, Bb, Cw, Cb, Dw, Db, Ew, Eb,
           bnh_g, bnh_b, bne_g, bne_b, W0, b0, W1, b1, W2, b2):
    f32 = jnp.float32
    atom_tabs = [atom_emb_0, atom_emb_1, atom_emb_2, atom_emb_3, atom_emb_4,
                 atom_emb_5, atom_emb_6, atom_emb_7, atom_emb_8]
    acat = jnp.stack([jnp.pad(t, ((0, 128 - t.shape[0]), (0, 0)))
                      for t in atom_tabs])
    bond_tabs = [bond_emb_0, bond_emb_1, bond_emb_2]
    bcat = jnp.stack([jnp.pad(t, ((0, 8 - t.shape[0]), (0, 0)))
                      for t in bond_tabs])
    r1 = lambda x: x.reshape(1, -1).astype(f32)

    src = edge_index[0].astype(jnp.int32)
    dst = edge_index[1].astype(jnp.int32)
    src2 = jnp.concatenate([src, src + NN])
    src2r = src2.reshape(2, NS, NCHUNK, EC)
    dstr = jnp.broadcast_to(dst.reshape(1, NS, NCHUNK, EC),
                            (2, NS, NCHUNK, EC))
    icat = jnp.stack([src2r, dstr], axis=3).reshape(2 * NS * NCHUNK, 2, EC)

    h = h.astype(jnp.int32)
    e = e.astype(jnp.int32)

    hf, ah, bd, eh = _enc_node_call(
        h, p.astype(f32), acat, Pw, r1(Pb),
        Aw[0], r1(Ab[0]), Bw[0], r1(Bb[0]), Dw[0], r1(Db[0]), Ew[0], r1(Eb[0]))
    ef, ce2 = _bond_edge_call(e, bcat, Cw[0], r1(Cb[0]))

    for l in range(NLAYER):
        last = (l == NLAYER - 1)
        bdtab = bd.reshape(2 * NN, 128)
        if last:
            (nd,) = _sc_edge_pass(bdtab, eh, ce2, icat, True)
        else:
            e2, nd = _sc_edge_pass(bdtab, eh, ce2, icat, False)
        hn, ps = _hnew_call(nd, ah)
        if last:
            (y,) = _final_call(hn, hf, ps, r1(bnh_g[l]), r1(bnh_b[l]),
                               W0, r1(b0), W1, r1(b1), W2, r1(b2))
        else:
            hf, ah, bd, eh = _nodeup_call(
                hn, hf, ps, r1(bnh_g[l]), r1(bnh_b[l]),
                Aw[l + 1], r1(Ab[l + 1]), Bw[l + 1], r1(Bb[l + 1]),
                Dw[l + 1], r1(Db[l + 1]), Ew[l + 1], r1(Eb[l + 1]))
            (est,) = _estats_call(e2)
            if l < NLAYER - 2:
                ef, ce2 = _edgeup_call(e2, ef, est, r1(bne_g[l]),
                                       r1(bne_b[l]), Cw[l + 1], r1(Cb[l + 1]),
                                       True)
            else:
                (ce2,) = _edgeup_call(e2, ef, est, r1(bne_g[l]),
                                      r1(bne_b[l]), Cw[l + 1], r1(Cb[l + 1]),
                                      False)
    return y
